# Initial kernel scaffold; baseline (speedup 1.0000x reference)
import jax
import jax.numpy as jnp
from jax.experimental import pallas as pl


def kernel(x, ln1_g, ln1_b, Wq, Wk, Wv, Wck, Wcv, Wg, Wo, ln2_g, ln2_b, W1, b1, W2, b2):
    raise NotImplementedError("write your pallas kernel here")


# speedup vs baseline: 9.1524x; 9.1524x over previous
import functools

import jax
import jax.numpy as jnp
from jax.experimental import pallas as pl
from jax.experimental.pallas import tpu as pltpu

B, S, D = 1, 2048, 768
H, DH = 12, 64
CBS = 32
SBS = 32
NSEL = 4
SW = 128
MLP = 3072
NB = S // CBS
SCALE = DH ** -0.5
NEG = -1e30

TQ = 256
NQT = S // TQ


def _ln_body(xt, g, b):
    mu = jnp.mean(xt, axis=-1, keepdims=True)
    xc = xt - mu
    var = jnp.mean(xc * xc, axis=-1, keepdims=True)
    return xc * jax.lax.rsqrt(var + 1e-5) * g + b


def _k1(x_ref, g_ref, b_ref, w_ref, o_ref):
    xn = _ln_body(x_ref[...], g_ref[...], b_ref[...])
    o_ref[...] = jnp.dot(xn, w_ref[...], preferred_element_type=jnp.float32)


def _proj(x, ln_g, ln_b, Wqkvg):
    return pl.pallas_call(
        _k1,
        grid=(NQT,),
        in_specs=[
            pl.BlockSpec((TQ, D), lambda i: (i, 0)),
            pl.BlockSpec((1, D), lambda i: (0, 0)),
            pl.BlockSpec((1, D), lambda i: (0, 0)),
            pl.BlockSpec((D, 3 * D + 3 * H), lambda i: (0, 0)),
        ],
        out_specs=pl.BlockSpec((TQ, 3 * D + 3 * H), lambda i: (i, 0)),
        out_shape=jax.ShapeDtypeStruct((S, 3 * D + 3 * H), jnp.float32),
    )(x, ln_g, ln_b, Wqkvg)


def _k2(q_ref, k_ref, v_ref, wck_ref, wcv_ref, ocmp_ref, selm_ref):
    q = q_ref[0]
    k = k_ref[0]
    v = v_ref[0]
    n_i = jax.lax.broadcasted_iota(jnp.int32, (NB, S), 0)
    s_i = jax.lax.broadcasted_iota(jnp.int32, (NB, S), 1)
    M = jnp.where(s_i // CBS == n_i, 1.0 / CBS, 0.0)
    kc = jnp.dot(jnp.dot(M, k, preferred_element_type=jnp.float32),
                 wck_ref[...], preferred_element_type=jnp.float32)
    vc = jnp.dot(jnp.dot(M, v, preferred_element_type=jnp.float32),
                 wcv_ref[...], preferred_element_type=jnp.float32)
    sc = jnp.dot(q, kc.T, preferred_element_type=jnp.float32) * SCALE
    pos = jax.lax.broadcasted_iota(jnp.int32, (S, NB), 0)
    blk = jax.lax.broadcasted_iota(jnp.int32, (S, NB), 1)
    cmask = (blk + 1) * CBS - 1 <= pos
    scm = jnp.where(cmask, sc, NEG)
    m = jnp.max(scm, axis=-1, keepdims=True)
    e = jnp.exp(scm - m)
    pc = e / jnp.sum(e, axis=-1, keepdims=True)
    cvalid = pos[:, :1] >= CBS - 1
    pc = jnp.where(cvalid, pc, 0.0)
    ocmp_ref[0] = jnp.dot(pc, vc, preferred_element_type=jnp.float32)
    imp = jnp.where(cmask, pc, -1.0)
    sel = jnp.zeros((S, NB), jnp.bool_)
    for _ in range(NSEL):
        mx = jnp.max(imp, axis=-1, keepdims=True)
        ismax = imp == mx
        first = jnp.min(jnp.where(ismax, blk, NB), axis=-1, keepdims=True)
        onehot = blk == first
        sel = sel | onehot
        imp = jnp.where(onehot, -2.0, imp)
    selm_ref[0] = sel.astype(jnp.float32)


def _compressed(q, k, v, Wck, Wcv):
    return pl.pallas_call(
        _k2,
        grid=(H,),
        in_specs=[
            pl.BlockSpec((1, S, DH), lambda h: (h, 0, 0)),
            pl.BlockSpec((1, S, DH), lambda h: (h, 0, 0)),
            pl.BlockSpec((1, S, DH), lambda h: (h, 0, 0)),
            pl.BlockSpec((DH, DH), lambda h: (0, 0)),
            pl.BlockSpec((DH, DH), lambda h: (0, 0)),
        ],
        out_specs=[
            pl.BlockSpec((1, S, DH), lambda h: (h, 0, 0)),
            pl.BlockSpec((1, S, NB), lambda h: (h, 0, 0)),
        ],
        out_shape=[
            jax.ShapeDtypeStruct((H, S, DH), jnp.float32),
            jax.ShapeDtypeStruct((H, S, NB), jnp.float32),
        ],
    )(q, k, v, Wck, Wcv)


def _k3(q_ref, k_ref, v_ref, selm_ref, ocmp_ref, g_ref, o_ref):
    i = pl.program_id(1)
    q = q_ref[0] * SCALE
    k = k_ref[0]
    v = v_ref[0]
    s = jnp.dot(q, k.T, preferred_element_type=jnp.float32)
    row = jax.lax.broadcasted_iota(jnp.int32, (TQ, S), 0) + i * TQ
    col = jax.lax.broadcasted_iota(jnp.int32, (TQ, S), 1)
    causal = col <= row
    wm = causal & (col > row - SW)
    sw = jnp.where(wm, s, NEG)
    mw = jnp.max(sw, axis=-1, keepdims=True)
    ew = jnp.exp(sw - mw)
    pw = ew / jnp.sum(ew, axis=-1, keepdims=True)
    o_win = jnp.dot(pw, v, preferred_element_type=jnp.float32)
    nb_i = jax.lax.broadcasted_iota(jnp.int32, (NB, S), 0)
    sp_i = jax.lax.broadcasted_iota(jnp.int32, (NB, S), 1)
    E = jnp.where(sp_i // SBS == nb_i, 1.0, 0.0)
    msel = jnp.dot(selm_ref[0], E, preferred_element_type=jnp.float32)
    sm = (msel > 0.5) & causal
    ss = jnp.where(sm, s, NEG)
    ms = jnp.max(ss, axis=-1, keepdims=True)
    es = jnp.exp(ss - ms)
    ps = es / jnp.sum(es, axis=-1, keepdims=True)
    o_sel = jnp.dot(ps, v, preferred_element_type=jnp.float32)
    g = jax.nn.sigmoid(g_ref[0])
    o_ref[0] = (g[:, 0:1] * ocmp_ref[0] + g[:, 1:2] * o_sel
                + g[:, 2:3] * o_win)


def _attention(q, k, v, selm, ocmp, gates):
    return pl.pallas_call(
        _k3,
        grid=(H, NQT),
        in_specs=[
            pl.BlockSpec((1, TQ, DH), lambda h, i: (h, i, 0)),
            pl.BlockSpec((1, S, DH), lambda h, i: (h, 0, 0)),
            pl.BlockSpec((1, S, DH), lambda h, i: (h, 0, 0)),
            pl.BlockSpec((1, TQ, NB), lambda h, i: (h, i, 0)),
            pl.BlockSpec((1, TQ, DH), lambda h, i: (h, i, 0)),
            pl.BlockSpec((1, TQ, 3), lambda h, i: (h, i, 0)),
        ],
        out_specs=pl.BlockSpec((1, TQ, DH), lambda h, i: (h, i, 0)),
        out_shape=jax.ShapeDtypeStruct((H, S, DH), jnp.float32),
    )(q, k, v, selm, ocmp, gates)


def _k4(o_ref, x_ref, wo_ref, y_ref):
    y_ref[...] = x_ref[...] + jnp.dot(o_ref[...], wo_ref[...],
                                      preferred_element_type=jnp.float32)


def _out_proj(o, x, Wo):
    return pl.pallas_call(
        _k4,
        grid=(NQT,),
        in_specs=[
            pl.BlockSpec((TQ, D), lambda i: (i, 0)),
            pl.BlockSpec((TQ, D), lambda i: (i, 0)),
            pl.BlockSpec((D, D), lambda i: (0, 0)),
        ],
        out_specs=pl.BlockSpec((TQ, D), lambda i: (i, 0)),
        out_shape=jax.ShapeDtypeStruct((S, D), jnp.float32),
    )(o, x, Wo)


def _k5(x_ref, g_ref, b_ref, w1_ref, b1_ref, w2_ref, b2_ref, y_ref):
    xt = x_ref[...]
    xn = _ln_body(xt, g_ref[...], b_ref[...])
    h = jax.nn.gelu(jnp.dot(xn, w1_ref[...], preferred_element_type=jnp.float32)
                    + b1_ref[...])
    y_ref[...] = xt + jnp.dot(h, w2_ref[...],
                              preferred_element_type=jnp.float32) + b2_ref[...]


def _ffn(x, ln_g, ln_b, W1, b1, W2, b2):
    return pl.pallas_call(
        _k5,
        grid=(NQT,),
        in_specs=[
            pl.BlockSpec((TQ, D), lambda i: (i, 0)),
            pl.BlockSpec((1, D), lambda i: (0, 0)),
            pl.BlockSpec((1, D), lambda i: (0, 0)),
            pl.BlockSpec((D, MLP), lambda i: (0, 0)),
            pl.BlockSpec((1, MLP), lambda i: (0, 0)),
            pl.BlockSpec((MLP, D), lambda i: (0, 0)),
            pl.BlockSpec((1, D), lambda i: (0, 0)),
        ],
        out_specs=pl.BlockSpec((TQ, D), lambda i: (i, 0)),
        out_shape=jax.ShapeDtypeStruct((S, D), jnp.float32),
    )(x, ln_g, ln_b, W1, b1, W2, b2)


@jax.jit
def _run(x, ln1_g, ln1_b, Wq, Wk, Wv, Wck, Wcv, Wg, Wo, ln2_g, ln2_b, W1, b1, W2, b2):
    x2 = x[0]
    Wqkvg = jnp.concatenate([Wq, Wk, Wv, Wg], axis=1)
    qkvg = _proj(x2, ln1_g[None], ln1_b[None], Wqkvg)
    q = qkvg[:, :D].reshape(S, H, DH).transpose(1, 0, 2)
    k = qkvg[:, D:2 * D].reshape(S, H, DH).transpose(1, 0, 2)
    v = qkvg[:, 2 * D:3 * D].reshape(S, H, DH).transpose(1, 0, 2)
    gates = qkvg[:, 3 * D:].reshape(S, H, 3).transpose(1, 0, 2)
    ocmp, selm = _compressed(q, k, v, Wck, Wcv)
    o = _attention(q, k, v, selm, ocmp, gates)
    o2 = o.transpose(1, 0, 2).reshape(S, D)
    x1 = _out_proj(o2, x2, Wo)
    y = _ffn(x1, ln2_g[None], ln2_b[None], W1, b1[None], W2, b2[None])
    return y[None]


def kernel(x, ln1_g, ln1_b, Wq, Wk, Wv, Wck, Wcv, Wg, Wo, ln2_g, ln2_b, W1, b1, W2, b2):
    return _run(x, ln1_g, ln1_b, Wq, Wk, Wv, Wck, Wcv, Wg, Wo,
                ln2_g, ln2_b, W1, b1, W2, b2)


# speedup vs baseline: 9.7780x; 1.0684x over previous
import jax
import jax.numpy as jnp
from jax.experimental import pallas as pl

B, S, D = 1, 2048, 768
H, DH = 12, 64
CBS = 32
SBS = 32
NSEL = 4
SW = 128
MLP = 3072
NB = S // CBS
SCALE = DH ** -0.5
NEG = -1e30

TQ = 256
NQT = S // TQ

F32 = jnp.float32
BF16 = jnp.bfloat16


def _ln_body(xt, g, b):
    mu = jnp.mean(xt, axis=-1, keepdims=True)
    xc = xt - mu
    var = jnp.mean(xc * xc, axis=-1, keepdims=True)
    return xc * jax.lax.rsqrt(var + 1e-5) * g + b


def _dot(a, b):
    return jnp.dot(a.astype(BF16), b.astype(BF16), preferred_element_type=F32)


def _k1(x_ref, g_ref, b_ref, w_ref, qkv_ref, gates_ref):
    xn = _ln_body(x_ref[...], g_ref[...], b_ref[...])
    res = _dot(xn, w_ref[...])
    qkv_ref[...] = res[:, :3 * D].astype(BF16)
    gates_ref[...] = res[:, 3 * D:]


def _proj(x, ln_g, ln_b, Wqkvg):
    return pl.pallas_call(
        _k1,
        grid=(NQT,),
        in_specs=[
            pl.BlockSpec((TQ, D), lambda i: (i, 0)),
            pl.BlockSpec((1, D), lambda i: (0, 0)),
            pl.BlockSpec((1, D), lambda i: (0, 0)),
            pl.BlockSpec((D, 3 * D + 3 * H), lambda i: (0, 0)),
        ],
        out_specs=[
            pl.BlockSpec((TQ, 3 * D), lambda i: (i, 0)),
            pl.BlockSpec((TQ, 3 * H), lambda i: (i, 0)),
        ],
        out_shape=[
            jax.ShapeDtypeStruct((S, 3 * D), BF16),
            jax.ShapeDtypeStruct((S, 3 * H), F32),
        ],
    )(x, ln_g, ln_b, Wqkvg)


def _k2(q_ref, k_ref, v_ref, wck_ref, wcv_ref, ocmp_ref, selm_ref):
    q = q_ref[0]
    k = k_ref[0]
    v = v_ref[0]
    n_i = jax.lax.broadcasted_iota(jnp.int32, (NB, S), 0)
    s_i = jax.lax.broadcasted_iota(jnp.int32, (NB, S), 1)
    M = jnp.where(s_i // CBS == n_i, 1.0 / CBS, 0.0)
    kc = _dot(_dot(M, k), wck_ref[...])
    vc = _dot(_dot(M, v), wcv_ref[...])
    sc = _dot(q, kc.T) * SCALE
    pos = jax.lax.broadcasted_iota(jnp.int32, (S, NB), 0)
    blk = jax.lax.broadcasted_iota(jnp.int32, (S, NB), 1)
    cmask = (blk + 1) * CBS - 1 <= pos
    scm = jnp.where(cmask, sc, NEG)
    m = jnp.max(scm, axis=-1, keepdims=True)
    e = jnp.exp(scm - m)
    pc = e / jnp.sum(e, axis=-1, keepdims=True)
    cvalid = pos[:, :1] >= CBS - 1
    pc = jnp.where(cvalid, pc, 0.0)
    ocmp_ref[0] = _dot(pc, vc)
    imp = jnp.where(cmask, pc, -1.0)
    sel = jnp.zeros((S, NB), jnp.bool_)
    for _ in range(NSEL):
        mx = jnp.max(imp, axis=-1, keepdims=True)
        ismax = imp == mx
        first = jnp.min(jnp.where(ismax, blk, NB), axis=-1, keepdims=True)
        onehot = blk == first
        sel = sel | onehot
        imp = jnp.where(onehot, -2.0, imp)
    selm_ref[0] = sel.astype(BF16)


def _compressed(q, k, v, Wck, Wcv):
    return pl.pallas_call(
        _k2,
        grid=(H,),
        in_specs=[
            pl.BlockSpec((1, S, DH), lambda h: (h, 0, 0)),
            pl.BlockSpec((1, S, DH), lambda h: (h, 0, 0)),
            pl.BlockSpec((1, S, DH), lambda h: (h, 0, 0)),
            pl.BlockSpec((DH, DH), lambda h: (0, 0)),
            pl.BlockSpec((DH, DH), lambda h: (0, 0)),
        ],
        out_specs=[
            pl.BlockSpec((1, S, DH), lambda h: (h, 0, 0)),
            pl.BlockSpec((1, S, NB), lambda h: (h, 0, 0)),
        ],
        out_shape=[
            jax.ShapeDtypeStruct((H, S, DH), F32),
            jax.ShapeDtypeStruct((H, S, NB), BF16),
        ],
    )(q, k, v, Wck, Wcv)


def _k3(q_ref, k_ref, v_ref, selm_ref, ocmp_ref, g_ref, o_ref):
    i = pl.program_id(1)
    q = q_ref[0]
    k = k_ref[0]
    v = v_ref[0]
    s = jnp.dot(q, k.T, preferred_element_type=F32) * SCALE
    row = jax.lax.broadcasted_iota(jnp.int32, (TQ, S), 0) + i * TQ
    col = jax.lax.broadcasted_iota(jnp.int32, (TQ, S), 1)
    causal = col <= row
    wm = causal & (col > row - SW)
    sw = jnp.where(wm, s, NEG)
    mw = jnp.max(sw, axis=-1, keepdims=True)
    ew = jnp.exp(sw - mw)
    pw = ew / jnp.sum(ew, axis=-1, keepdims=True)
    o_win = _dot(pw, v)
    nb_i = jax.lax.broadcasted_iota(jnp.int32, (NB, S), 0)
    sp_i = jax.lax.broadcasted_iota(jnp.int32, (NB, S), 1)
    E = jnp.where(sp_i // SBS == nb_i, 1.0, 0.0).astype(BF16)
    msel = jnp.dot(selm_ref[0], E, preferred_element_type=F32)
    sm = (msel > 0.5) & causal
    ss = jnp.where(sm, s, NEG)
    ms = jnp.max(ss, axis=-1, keepdims=True)
    es = jnp.exp(ss - ms)
    ps = es / jnp.sum(es, axis=-1, keepdims=True)
    o_sel = _dot(ps, v)
    g = jax.nn.sigmoid(g_ref[0])
    o_ref[0] = (g[:, 0:1] * ocmp_ref[0] + g[:, 1:2] * o_sel
                + g[:, 2:3] * o_win).astype(BF16)


def _attention(q, k, v, selm, ocmp, gates):
    return pl.pallas_call(
        _k3,
        grid=(H, NQT),
        in_specs=[
            pl.BlockSpec((1, TQ, DH), lambda h, i: (h, i, 0)),
            pl.BlockSpec((1, S, DH), lambda h, i: (h, 0, 0)),
            pl.BlockSpec((1, S, DH), lambda h, i: (h, 0, 0)),
            pl.BlockSpec((1, TQ, NB), lambda h, i: (h, i, 0)),
            pl.BlockSpec((1, TQ, DH), lambda h, i: (h, i, 0)),
            pl.BlockSpec((1, TQ, 3), lambda h, i: (h, i, 0)),
        ],
        out_specs=pl.BlockSpec((1, TQ, DH), lambda h, i: (h, i, 0)),
        out_shape=jax.ShapeDtypeStruct((H, S, DH), BF16),
    )(q, k, v, selm, ocmp, gates)


def _k4(o_ref, x_ref, wo_ref, y_ref):
    y_ref[...] = x_ref[...] + jnp.dot(o_ref[...], wo_ref[...],
                                      preferred_element_type=F32)


def _out_proj(o, x, Wo):
    return pl.pallas_call(
        _k4,
        grid=(NQT,),
        in_specs=[
            pl.BlockSpec((TQ, D), lambda i: (i, 0)),
            pl.BlockSpec((TQ, D), lambda i: (i, 0)),
            pl.BlockSpec((D, D), lambda i: (0, 0)),
        ],
        out_specs=pl.BlockSpec((TQ, D), lambda i: (i, 0)),
        out_shape=jax.ShapeDtypeStruct((S, D), F32),
    )(o, x, Wo)


def _k5(x_ref, g_ref, b_ref, w1_ref, b1_ref, w2_ref, b2_ref, y_ref):
    xt = x_ref[...]
    xn = _ln_body(xt, g_ref[...], b_ref[...])
    h = jax.nn.gelu(_dot(xn, w1_ref[...]) + b1_ref[...])
    y_ref[...] = xt + _dot(h, w2_ref[...]) + b2_ref[...]


def _ffn(x, ln_g, ln_b, W1, b1, W2, b2):
    return pl.pallas_call(
        _k5,
        grid=(NQT,),
        in_specs=[
            pl.BlockSpec((TQ, D), lambda i: (i, 0)),
            pl.BlockSpec((1, D), lambda i: (0, 0)),
            pl.BlockSpec((1, D), lambda i: (0, 0)),
            pl.BlockSpec((D, MLP), lambda i: (0, 0)),
            pl.BlockSpec((1, MLP), lambda i: (0, 0)),
            pl.BlockSpec((MLP, D), lambda i: (0, 0)),
            pl.BlockSpec((1, D), lambda i: (0, 0)),
        ],
        out_specs=pl.BlockSpec((TQ, D), lambda i: (i, 0)),
        out_shape=jax.ShapeDtypeStruct((S, D), F32),
    )(x, ln_g, ln_b, W1, b1, W2, b2)


@jax.jit
def _run(x, ln1_g, ln1_b, Wq, Wk, Wv, Wck, Wcv, Wg, Wo, ln2_g, ln2_b, W1, b1, W2, b2):
    x2 = x[0]
    Wqkvg = jnp.concatenate([Wq, Wk, Wv, Wg], axis=1)
    qkv, gates = _proj(x2, ln1_g[None], ln1_b[None], Wqkvg)
    q = qkv[:, :D].reshape(S, H, DH).transpose(1, 0, 2)
    k = qkv[:, D:2 * D].reshape(S, H, DH).transpose(1, 0, 2)
    v = qkv[:, 2 * D:3 * D].reshape(S, H, DH).transpose(1, 0, 2)
    gates_h = gates.reshape(S, H, 3).transpose(1, 0, 2)
    ocmp, selm = _compressed(q, k, v, Wck, Wcv)
    o = _attention(q, k, v, selm, ocmp, gates_h)
    o2 = o.transpose(1, 0, 2).reshape(S, D)
    x1 = _out_proj(o2, x2, Wo)
    y = _ffn(x1, ln2_g[None], ln2_b[None], W1, b1[None], W2, b2[None])
    return y[None]


def kernel(x, ln1_g, ln1_b, Wq, Wk, Wv, Wck, Wcv, Wg, Wo, ln2_g, ln2_b, W1, b1, W2, b2):
    return _run(x, ln1_g, ln1_b, Wq, Wk, Wv, Wck, Wcv, Wg, Wo,
                ln2_g, ln2_b, W1, b1, W2, b2)


# speedup vs baseline: 12.6768x; 1.2965x over previous
import jax
import jax.numpy as jnp
from jax.experimental import pallas as pl

B, S, D = 1, 2048, 768
H, DH = 12, 64
CBS = 32
SBS = 32
NSEL = 4
SW = 128
MLP = 3072
NB = S // CBS
SCALE = DH ** -0.5
NEG = -1e30

TQ = 256
NQT = S // TQ

F32 = jnp.float32
BF16 = jnp.bfloat16


def _ln_body(xt, g, b):
    mu = jnp.mean(xt, axis=-1, keepdims=True)
    xc = xt - mu
    var = jnp.mean(xc * xc, axis=-1, keepdims=True)
    return xc * jax.lax.rsqrt(var + 1e-5) * g + b


def _dot(a, b):
    return jnp.dot(a.astype(BF16), b.astype(BF16), preferred_element_type=F32)


def _k1(x_ref, g_ref, b_ref, w_ref, qkv_ref, gates_ref):
    xn = _ln_body(x_ref[...], g_ref[...], b_ref[...])
    res = _dot(xn, w_ref[...])
    qkv_ref[...] = res[:, :3 * D].astype(BF16)
    gates_ref[...] = res[:, 3 * D:]


def _proj(x, ln_g, ln_b, Wqkvg):
    return pl.pallas_call(
        _k1,
        grid=(NQT,),
        in_specs=[
            pl.BlockSpec((TQ, D), lambda i: (i, 0)),
            pl.BlockSpec((1, D), lambda i: (0, 0)),
            pl.BlockSpec((1, D), lambda i: (0, 0)),
            pl.BlockSpec((D, 3 * D + 3 * H), lambda i: (0, 0)),
        ],
        out_specs=[
            pl.BlockSpec((TQ, 3 * D), lambda i: (i, 0)),
            pl.BlockSpec((TQ, 3 * H), lambda i: (i, 0)),
        ],
        out_shape=[
            jax.ShapeDtypeStruct((S, 3 * D), BF16),
            jax.ShapeDtypeStruct((S, 3 * H), F32),
        ],
    )(x, ln_g, ln_b, Wqkvg)


def _k2(q_ref, k_ref, v_ref, wck_ref, wcv_ref, ocmp_ref, selm_ref):
    q = q_ref[0]
    k = k_ref[0]
    v = v_ref[0][:, :DH]
    n_i = jax.lax.broadcasted_iota(jnp.int32, (NB, S), 0)
    s_i = jax.lax.broadcasted_iota(jnp.int32, (NB, S), 1)
    M = jnp.where(s_i // CBS == n_i, 1.0 / CBS, 0.0)
    kc = _dot(_dot(M, k), wck_ref[...])
    vc = _dot(_dot(M, v), wcv_ref[...])
    sc = _dot(q, kc.T) * SCALE
    pos = jax.lax.broadcasted_iota(jnp.int32, (S, NB), 0)
    blk = jax.lax.broadcasted_iota(jnp.int32, (S, NB), 1)
    cmask = (blk + 1) * CBS - 1 <= pos
    scm = jnp.where(cmask, sc, NEG)
    m = jnp.max(scm, axis=-1, keepdims=True)
    e = jnp.exp(scm - m)
    pc = e / jnp.sum(e, axis=-1, keepdims=True)
    cvalid = pos[:, :1] >= CBS - 1
    pc = jnp.where(cvalid, pc, 0.0)
    ocmp_ref[0] = _dot(pc, vc)
    imp = jnp.where(cmask, pc, -1.0)
    sel = jnp.zeros((S, NB), jnp.bool_)
    for _ in range(NSEL):
        mx = jnp.max(imp, axis=-1, keepdims=True)
        ismax = imp == mx
        first = jnp.min(jnp.where(ismax, blk, NB), axis=-1, keepdims=True)
        onehot = blk == first
        sel = sel | onehot
        imp = jnp.where(onehot, -2.0, imp)
    selm_ref[0] = sel.astype(BF16)


def _compressed(q, k, v, Wck, Wcv):
    return pl.pallas_call(
        _k2,
        grid=(H,),
        in_specs=[
            pl.BlockSpec((1, S, DH), lambda h: (h, 0, 0)),
            pl.BlockSpec((1, S, DH), lambda h: (h, 0, 0)),
            pl.BlockSpec((1, S, DH * 2), lambda h: (h, 0, 0)),
            pl.BlockSpec((DH, DH), lambda h: (0, 0)),
            pl.BlockSpec((DH, DH), lambda h: (0, 0)),
        ],
        out_specs=[
            pl.BlockSpec((1, S, DH), lambda h: (h, 0, 0)),
            pl.BlockSpec((1, S, NB), lambda h: (h, 0, 0)),
        ],
        out_shape=[
            jax.ShapeDtypeStruct((H, S, DH), F32),
            jax.ShapeDtypeStruct((H, S, NB), BF16),
        ],
    )(q, k, v, Wck, Wcv)


def _k3(q_ref, k_ref, vaug_ref, selm_ref, ocmp_ref, g_ref, o_ref):
    i = pl.program_id(1)
    q = q_ref[0]
    k = k_ref[0]
    vaug = vaug_ref[0]
    s = jnp.dot(q, k.T, preferred_element_type=F32) * SCALE
    c = jnp.max(s, axis=-1, keepdims=True)
    e = jnp.exp(s - c)
    row = jax.lax.broadcasted_iota(jnp.int32, (TQ, S), 0) + i * TQ
    col = jax.lax.broadcasted_iota(jnp.int32, (TQ, S), 1)
    causal = col <= row
    nb_i = jax.lax.broadcasted_iota(jnp.int32, (NB, S), 0)
    sp_i = jax.lax.broadcasted_iota(jnp.int32, (NB, S), 1)
    Ex = jnp.where(sp_i // SBS == nb_i, 1.0, 0.0).astype(BF16)
    msel = jnp.dot(selm_ref[0], Ex, preferred_element_type=F32)
    es = jnp.where(causal, e * msel, 0.0).astype(BF16)
    oz = jnp.dot(es, vaug, preferred_element_type=F32)
    o_sel = oz[:, :DH] / oz[:, DH:DH + 1]
    start = jnp.maximum(i - 1, 0) * TQ
    kslab = k_ref[0, pl.ds(start, 2 * TQ), :]
    sslab = jnp.dot(q, kslab.T, preferred_element_type=F32) * SCALE
    eslab = jnp.exp(sslab - c)
    colw = jax.lax.broadcasted_iota(jnp.int32, (TQ, 2 * TQ), 1) + start
    roww = jax.lax.broadcasted_iota(jnp.int32, (TQ, 2 * TQ), 0) + i * TQ
    wm = (colw <= roww) & (colw > roww - SW)
    ew = jnp.where(wm, eslab, 0.0).astype(BF16)
    vslab = vaug_ref[0, pl.ds(start, 2 * TQ), :]
    wz = jnp.dot(ew, vslab, preferred_element_type=F32)
    o_win = wz[:, :DH] / wz[:, DH:DH + 1]
    g = jax.nn.sigmoid(g_ref[0])
    o_ref[0] = (g[:, 0:1] * ocmp_ref[0] + g[:, 1:2] * o_sel
                + g[:, 2:3] * o_win).astype(BF16)


def _attention(q, k, vaug, selm, ocmp, gates):
    return pl.pallas_call(
        _k3,
        grid=(H, NQT),
        in_specs=[
            pl.BlockSpec((1, TQ, DH), lambda h, i: (h, i, 0)),
            pl.BlockSpec((1, S, DH), lambda h, i: (h, 0, 0)),
            pl.BlockSpec((1, S, DH * 2), lambda h, i: (h, 0, 0)),
            pl.BlockSpec((1, TQ, NB), lambda h, i: (h, i, 0)),
            pl.BlockSpec((1, TQ, DH), lambda h, i: (h, i, 0)),
            pl.BlockSpec((1, TQ, 3), lambda h, i: (h, i, 0)),
        ],
        out_specs=pl.BlockSpec((1, TQ, DH), lambda h, i: (h, i, 0)),
        out_shape=jax.ShapeDtypeStruct((H, S, DH), BF16),
    )(q, k, vaug, selm, ocmp, gates)


def _k4(o_ref, x_ref, wo_ref, y_ref):
    y_ref[...] = x_ref[...] + jnp.dot(o_ref[...], wo_ref[...],
                                      preferred_element_type=F32)


def _out_proj(o, x, Wo):
    return pl.pallas_call(
        _k4,
        grid=(NQT,),
        in_specs=[
            pl.BlockSpec((TQ, D), lambda i: (i, 0)),
            pl.BlockSpec((TQ, D), lambda i: (i, 0)),
            pl.BlockSpec((D, D), lambda i: (0, 0)),
        ],
        out_specs=pl.BlockSpec((TQ, D), lambda i: (i, 0)),
        out_shape=jax.ShapeDtypeStruct((S, D), F32),
    )(o, x, Wo)


def _k5(x_ref, g_ref, b_ref, w1_ref, b1_ref, w2_ref, b2_ref, y_ref):
    xt = x_ref[...]
    xn = _ln_body(xt, g_ref[...], b_ref[...])
    h = jax.nn.gelu(_dot(xn, w1_ref[...]) + b1_ref[...])
    y_ref[...] = xt + _dot(h, w2_ref[...]) + b2_ref[...]


def _ffn(x, ln_g, ln_b, W1, b1, W2, b2):
    return pl.pallas_call(
        _k5,
        grid=(NQT,),
        in_specs=[
            pl.BlockSpec((TQ, D), lambda i: (i, 0)),
            pl.BlockSpec((1, D), lambda i: (0, 0)),
            pl.BlockSpec((1, D), lambda i: (0, 0)),
            pl.BlockSpec((D, MLP), lambda i: (0, 0)),
            pl.BlockSpec((1, MLP), lambda i: (0, 0)),
            pl.BlockSpec((MLP, D), lambda i: (0, 0)),
            pl.BlockSpec((1, D), lambda i: (0, 0)),
        ],
        out_specs=pl.BlockSpec((TQ, D), lambda i: (i, 0)),
        out_shape=jax.ShapeDtypeStruct((S, D), F32),
    )(x, ln_g, ln_b, W1, b1, W2, b2)


@jax.jit
def _run(x, ln1_g, ln1_b, Wq, Wk, Wv, Wck, Wcv, Wg, Wo, ln2_g, ln2_b, W1, b1, W2, b2):
    x2 = x[0]
    Wqkvg = jnp.concatenate([Wq, Wk, Wv, Wg], axis=1)
    qkv, gates = _proj(x2, ln1_g[None], ln1_b[None], Wqkvg)
    q = qkv[:, :D].reshape(S, H, DH).transpose(1, 0, 2)
    k = qkv[:, D:2 * D].reshape(S, H, DH).transpose(1, 0, 2)
    v = qkv[:, 2 * D:3 * D].reshape(S, H, DH).transpose(1, 0, 2)
    vaug = jnp.concatenate(
        [v, jnp.ones((H, S, 1), BF16), jnp.zeros((H, S, DH - 1), BF16)], -1)
    gates_h = gates.reshape(S, H, 3).transpose(1, 0, 2)
    ocmp, selm = _compressed(q, k, vaug, Wck, Wcv)
    o = _attention(q, k, vaug, selm, ocmp, gates_h)
    o2 = o.transpose(1, 0, 2).reshape(S, D)
    x1 = _out_proj(o2, x2, Wo)
    y = _ffn(x1, ln2_g[None], ln2_b[None], W1, b1[None], W2, b2[None])
    return y[None]


def kernel(x, ln1_g, ln1_b, Wq, Wk, Wv, Wck, Wcv, Wg, Wo, ln2_g, ln2_b, W1, b1, W2, b2):
    return _run(x, ln1_g, ln1_b, Wq, Wk, Wv, Wck, Wcv, Wg, Wo,
                ln2_g, ln2_b, W1, b1, W2, b2)


# speedup vs baseline: 15.6416x; 1.2339x over previous
import functools

import jax
import jax.numpy as jnp
from jax.experimental import pallas as pl

B, S, D = 1, 2048, 768
H, DH = 12, 64
CBS = 32
SBS = 32
NSEL = 4
SW = 128
MLP = 3072
NB = S // CBS
SCALE = DH ** -0.5
NEG = -1e30

TQ = 256
NQT = S // TQ

F32 = jnp.float32
BF16 = jnp.bfloat16


def _ln_body(xt, g, b):
    mu = jnp.mean(xt, axis=-1, keepdims=True)
    xc = xt - mu
    var = jnp.mean(xc * xc, axis=-1, keepdims=True)
    return xc * jax.lax.rsqrt(var + 1e-5) * g + b


def _dot(a, b):
    return jnp.dot(a.astype(BF16), b.astype(BF16), preferred_element_type=F32)


def _dot_tlhs(a, b):
    return jax.lax.dot_general(a.astype(BF16), b.astype(BF16),
                               (((0,), (0,)), ((), ())),
                               preferred_element_type=F32)


def _k1(x_ref, g_ref, b_ref, wq_ref, wk_ref, wv_ref, wg_ref,
        q_ref, k_ref, v_ref, gates_ref):
    xn = _ln_body(x_ref[...], g_ref[...], b_ref[...])
    q_ref[...] = _dot(xn, wq_ref[...]).astype(BF16)
    k_ref[...] = _dot(xn, wk_ref[...]).astype(BF16)
    v_ref[...] = _dot(xn, wv_ref[...]).astype(BF16)
    gates_ref[...] = _dot(xn, wg_ref[...])


def _proj(x, ln_g, ln_b, Wq, Wk, Wv, Wg):
    return pl.pallas_call(
        _k1,
        grid=(NQT,),
        in_specs=[
            pl.BlockSpec((TQ, D), lambda i: (i, 0)),
            pl.BlockSpec((1, D), lambda i: (0, 0)),
            pl.BlockSpec((1, D), lambda i: (0, 0)),
            pl.BlockSpec((D, D), lambda i: (0, 0)),
            pl.BlockSpec((D, D), lambda i: (0, 0)),
            pl.BlockSpec((D, D), lambda i: (0, 0)),
            pl.BlockSpec((D, 3 * H), lambda i: (0, 0)),
        ],
        out_specs=[
            pl.BlockSpec((TQ, D), lambda i: (i, 0)),
            pl.BlockSpec((TQ, D), lambda i: (i, 0)),
            pl.BlockSpec((TQ, D), lambda i: (i, 0)),
            pl.BlockSpec((TQ, 3 * H), lambda i: (i, 0)),
        ],
        out_shape=[
            jax.ShapeDtypeStruct((S, D), BF16),
            jax.ShapeDtypeStruct((S, D), BF16),
            jax.ShapeDtypeStruct((S, D), BF16),
            jax.ShapeDtypeStruct((S, 3 * H), F32),
        ],
    )(x, ln_g, ln_b, Wq, Wk, Wv, Wg)


def _k2(q_ref, k_ref, v_ref, wck_ref, wcv_ref, ocmp_ref, selm_ref):
    q = q_ref[0]
    k = k_ref[0]
    v = v_ref[0][:, :DH]
    n_i = jax.lax.broadcasted_iota(jnp.int32, (NB, S), 0)
    s_i = jax.lax.broadcasted_iota(jnp.int32, (NB, S), 1)
    M = jnp.where(s_i // CBS == n_i, 1.0 / CBS, 0.0)
    kc = _dot(_dot(M, k), wck_ref[...])
    vc = _dot(_dot(M, v), wcv_ref[...])
    scT = jax.lax.dot_general(kc.astype(BF16), q.astype(BF16),
                              (((1,), (1,)), ((), ())),
                              preferred_element_type=F32) * SCALE
    posT = s_i
    blkT = n_i
    cmaskT = (blkT + 1) * CBS - 1 <= posT
    scmT = jnp.where(cmaskT, scT, NEG)
    m = jnp.max(scmT, axis=0, keepdims=True)
    e = jnp.exp(scmT - m)
    pcT = e / jnp.sum(e, axis=0, keepdims=True)
    cvalidT = posT[:1] >= CBS - 1
    pcT = jnp.where(cvalidT, pcT, 0.0)
    ocmp_ref[0] = _dot_tlhs(pcT, vc)
    impT = jnp.where(cmaskT, pcT, -1.0)
    sel = jnp.zeros((NB, S), jnp.bool_)
    for _ in range(NSEL):
        mx = jnp.max(impT, axis=0, keepdims=True)
        ismax = impT == mx
        first = jnp.min(jnp.where(ismax, blkT, NB), axis=0, keepdims=True)
        onehot = blkT == first
        sel = sel | onehot
        impT = jnp.where(onehot, -2.0, impT)
    selm_ref[0] = sel.astype(BF16)


def _compressed(q, k, vaug, Wck, Wcv):
    return pl.pallas_call(
        _k2,
        grid=(H,),
        in_specs=[
            pl.BlockSpec((1, S, DH), lambda h: (h, 0, 0)),
            pl.BlockSpec((1, S, DH), lambda h: (h, 0, 0)),
            pl.BlockSpec((1, S, DH * 2), lambda h: (h, 0, 0)),
            pl.BlockSpec((DH, DH), lambda h: (0, 0)),
            pl.BlockSpec((DH, DH), lambda h: (0, 0)),
        ],
        out_specs=[
            pl.BlockSpec((1, S, DH), lambda h: (h, 0, 0)),
            pl.BlockSpec((1, NB, S), lambda h: (h, 0, 0)),
        ],
        out_shape=[
            jax.ShapeDtypeStruct((H, S, DH), F32),
            jax.ShapeDtypeStruct((H, NB, S), BF16),
        ],
    )(q, k, vaug, Wck, Wcv)


def _k3(q_ref, k_ref, vaug_ref, selm_ref, ocmp_ref, g_ref, o_ref, *, kw, i0):
    i = pl.program_id(1) + i0
    q = q_ref[0]
    k = k_ref[0]
    s = jnp.dot(q, k.T, preferred_element_type=F32) * SCALE
    c = jnp.max(s, axis=-1, keepdims=True)
    row = jax.lax.broadcasted_iota(jnp.int32, (TQ, kw), 0) + i * TQ
    col = jax.lax.broadcasted_iota(jnp.int32, (TQ, kw), 1)
    causal = col <= row
    nb_i = jax.lax.broadcasted_iota(jnp.int32, (NB, kw), 0)
    sp_i = jax.lax.broadcasted_iota(jnp.int32, (NB, kw), 1)
    Ex = jnp.where(sp_i // SBS == nb_i, 1.0, 0.0).astype(BF16)
    msel = _dot_tlhs(selm_ref[0], Ex)
    es = jnp.where(causal, jnp.exp(s - c) * msel, 0.0).astype(BF16)
    oz = jnp.dot(es, vaug_ref[0], preferred_element_type=F32)
    o_sel = oz[:, :DH] / oz[:, DH:DH + 1]
    start = jnp.maximum(i - 1, 0) * TQ
    kslab = k_ref[0, pl.ds(start, 2 * TQ), :]
    sslab = jnp.dot(q, kslab.T, preferred_element_type=F32) * SCALE
    eslab = jnp.exp(sslab - c)
    colw = jax.lax.broadcasted_iota(jnp.int32, (TQ, 2 * TQ), 1) + start
    roww = jax.lax.broadcasted_iota(jnp.int32, (TQ, 2 * TQ), 0) + i * TQ
    wm = (colw <= roww) & (colw > roww - SW)
    ew = jnp.where(wm, eslab, 0.0).astype(BF16)
    vslab = vaug_ref[0, pl.ds(start, 2 * TQ), :]
    wz = jnp.dot(ew, vslab, preferred_element_type=F32)
    o_win = wz[:, :DH] / wz[:, DH:DH + 1]
    g = jax.nn.sigmoid(g_ref[0])
    o_ref[0] = (g[:, 0:1] * ocmp_ref[0] + g[:, 1:2] * o_sel
                + g[:, 2:3] * o_win).astype(BF16)


def _attention_part(q, k, vaug, selm, ocmp, gates, kw, i0, nt):
    body = functools.partial(_k3, kw=kw, i0=i0)
    return pl.pallas_call(
        body,
        grid=(H, nt),
        in_specs=[
            pl.BlockSpec((1, TQ, DH), lambda h, i: (h, i + i0, 0)),
            pl.BlockSpec((1, kw, DH), lambda h, i: (h, 0, 0)),
            pl.BlockSpec((1, kw, DH * 2), lambda h, i: (h, 0, 0)),
            pl.BlockSpec((1, NB, TQ), lambda h, i: (h, 0, i + i0)),
            pl.BlockSpec((1, TQ, DH), lambda h, i: (h, i + i0, 0)),
            pl.BlockSpec((1, TQ, 3), lambda h, i: (h, i + i0, 0)),
        ],
        out_specs=pl.BlockSpec((1, TQ, DH), lambda h, i: (h, i, 0)),
        out_shape=jax.ShapeDtypeStruct((H, nt * TQ, DH), BF16),
    )(q, k, vaug, selm, ocmp, gates)


def _k4(o_ref, x_ref, wo_ref, y_ref):
    y_ref[...] = x_ref[...] + _dot(o_ref[...], wo_ref[...])


def _out_proj(o, x, Wo):
    return pl.pallas_call(
        _k4,
        grid=(NQT,),
        in_specs=[
            pl.BlockSpec((TQ, D), lambda i: (i, 0)),
            pl.BlockSpec((TQ, D), lambda i: (i, 0)),
            pl.BlockSpec((D, D), lambda i: (0, 0)),
        ],
        out_specs=pl.BlockSpec((TQ, D), lambda i: (i, 0)),
        out_shape=jax.ShapeDtypeStruct((S, D), F32),
    )(o, x, Wo)


def _k5(x_ref, g_ref, b_ref, w1_ref, b1_ref, w2_ref, b2_ref, y_ref):
    xt = x_ref[...]
    xn = _ln_body(xt, g_ref[...], b_ref[...])
    h = jax.nn.gelu(_dot(xn, w1_ref[...]) + b1_ref[...])
    y_ref[...] = xt + _dot(h, w2_ref[...]) + b2_ref[...]


def _ffn(x, ln_g, ln_b, W1, b1, W2, b2):
    return pl.pallas_call(
        _k5,
        grid=(NQT,),
        in_specs=[
            pl.BlockSpec((TQ, D), lambda i: (i, 0)),
            pl.BlockSpec((1, D), lambda i: (0, 0)),
            pl.BlockSpec((1, D), lambda i: (0, 0)),
            pl.BlockSpec((D, MLP), lambda i: (0, 0)),
            pl.BlockSpec((1, MLP), lambda i: (0, 0)),
            pl.BlockSpec((MLP, D), lambda i: (0, 0)),
            pl.BlockSpec((1, D), lambda i: (0, 0)),
        ],
        out_specs=pl.BlockSpec((TQ, D), lambda i: (i, 0)),
        out_shape=jax.ShapeDtypeStruct((S, D), F32),
    )(x, ln_g, ln_b, W1, b1, W2, b2)


@jax.jit
def _run(x, ln1_g, ln1_b, Wq, Wk, Wv, Wck, Wcv, Wg, Wo, ln2_g, ln2_b, W1, b1, W2, b2):
    x2 = x[0]
    qf, kf, vf, gates = _proj(x2, ln1_g[None], ln1_b[None], Wq, Wk, Wv, Wg)
    q = qf.reshape(S, H, DH).transpose(1, 0, 2)
    k = kf.reshape(S, H, DH).transpose(1, 0, 2)
    v = vf.reshape(S, H, DH).transpose(1, 0, 2)
    vaug = jnp.concatenate(
        [v, jnp.ones((H, S, 1), BF16), jnp.zeros((H, S, DH - 1), BF16)], -1)
    gates_h = gates.reshape(S, H, 3).transpose(1, 0, 2)
    ocmp, selm = _compressed(q, k, vaug, Wck, Wcv)
    oA = _attention_part(q, k, vaug, selm, ocmp, gates_h, S // 2, 0, NQT // 2)
    oB = _attention_part(q, k, vaug, selm, ocmp, gates_h, S, NQT // 2, NQT // 2)
    o = jnp.concatenate([oA, oB], axis=1)
    o2 = o.transpose(1, 0, 2).reshape(S, D)
    x1 = _out_proj(o2, x2, Wo)
    y = _ffn(x1, ln2_g[None], ln2_b[None], W1, b1[None], W2, b2[None])
    return y[None]


def kernel(x, ln1_g, ln1_b, Wq, Wk, Wv, Wck, Wcv, Wg, Wo, ln2_g, ln2_b, W1, b1, W2, b2):
    return _run(x, ln1_g, ln1_b, Wq, Wk, Wv, Wck, Wcv, Wg, Wo,
                ln2_g, ln2_b, W1, b1, W2, b2)


# speedup vs baseline: 16.6987x; 1.0676x over previous
import functools

import jax
import jax.numpy as jnp
from jax.experimental import pallas as pl

B, S, D = 1, 2048, 768
H, DH = 12, 64
CBS = 32
SBS = 32
NSEL = 4
SW = 128
MLP = 3072
NB = S // CBS
SCALE = DH ** -0.5
NEG = -1e30

TQ = 256
NQT = S // TQ
HP = H // 2

F32 = jnp.float32
BF16 = jnp.bfloat16


def _ln_body(xt, g, b):
    mu = jnp.mean(xt, axis=-1, keepdims=True)
    xc = xt - mu
    var = jnp.mean(xc * xc, axis=-1, keepdims=True)
    return xc * jax.lax.rsqrt(var + 1e-5) * g + b


def _dot(a, b):
    return jnp.dot(a.astype(BF16), b.astype(BF16), preferred_element_type=F32)


def _dot_tlhs(a, b):
    return jax.lax.dot_general(a.astype(BF16), b.astype(BF16),
                               (((0,), (0,)), ((), ())),
                               preferred_element_type=F32)


def _dot_trhs(a, b):
    return jax.lax.dot_general(a.astype(BF16), b.astype(BF16),
                               (((1,), (1,)), ((), ())),
                               preferred_element_type=F32)


def _k1(x_ref, g_ref, b_ref, wq_ref, wk_ref, wv_ref, wg_ref,
        q_ref, k_ref, v_ref, gates_ref):
    xn = _ln_body(x_ref[...], g_ref[...], b_ref[...])
    q_ref[...] = _dot(xn, wq_ref[...]).astype(BF16)
    k_ref[...] = _dot(xn, wk_ref[...]).astype(BF16)
    v_ref[...] = _dot(xn, wv_ref[...]).astype(BF16)
    gates_ref[...] = _dot(xn, wg_ref[...])


def _proj(x, ln_g, ln_b, Wq, Wk, Wv, Wg):
    return pl.pallas_call(
        _k1,
        grid=(NQT,),
        in_specs=[
            pl.BlockSpec((TQ, D), lambda i: (i, 0)),
            pl.BlockSpec((1, D), lambda i: (0, 0)),
            pl.BlockSpec((1, D), lambda i: (0, 0)),
            pl.BlockSpec((D, D), lambda i: (0, 0)),
            pl.BlockSpec((D, D), lambda i: (0, 0)),
            pl.BlockSpec((D, D), lambda i: (0, 0)),
            pl.BlockSpec((D, 3 * H), lambda i: (0, 0)),
        ],
        out_specs=[
            pl.BlockSpec((TQ, D), lambda i: (i, 0)),
            pl.BlockSpec((TQ, D), lambda i: (i, 0)),
            pl.BlockSpec((TQ, D), lambda i: (i, 0)),
            pl.BlockSpec((TQ, 3 * H), lambda i: (i, 0)),
        ],
        out_shape=[
            jax.ShapeDtypeStruct((S, D), BF16),
            jax.ShapeDtypeStruct((S, D), BF16),
            jax.ShapeDtypeStruct((S, D), BF16),
            jax.ShapeDtypeStruct((S, 3 * H), F32),
        ],
    )(x, ln_g, ln_b, Wq, Wk, Wv, Wg)


def _k3(q_ref, k_ref, v_ref, g_ref, wck_ref, wcv_ref, o_ref, *, kw, i0, nbk):
    i = pl.program_id(1) + i0
    q2 = q_ref[...]
    k2 = k_ref[...]
    v2 = v_ref[...]
    start = jnp.maximum(i - 1, 0) * TQ
    kslab2 = k_ref[pl.ds(start, 2 * TQ), :]
    vslab2 = v_ref[pl.ds(start, 2 * TQ), :]
    n_i = jax.lax.broadcasted_iota(jnp.int32, (nbk, kw), 0)
    s_i = jax.lax.broadcasted_iota(jnp.int32, (nbk, kw), 1)
    M = jnp.where(s_i // CBS == n_i, 1.0 / CBS, 0.0).astype(BF16)
    Ex = jnp.where(s_i // SBS == n_i, 1.0, 0.0).astype(BF16)
    posT = jax.lax.broadcasted_iota(jnp.int32, (nbk, TQ), 1) + i * TQ
    blkT = jax.lax.broadcasted_iota(jnp.int32, (nbk, TQ), 0)
    cmaskT = (blkT + 1) * CBS - 1 <= posT
    colw = jax.lax.broadcasted_iota(jnp.int32, (TQ, 2 * TQ), 1) + start
    roww = jax.lax.broadcasted_iota(jnp.int32, (TQ, 2 * TQ), 0) + i * TQ
    wm = (colw <= roww) & (colw > roww - SW)
    if i0 == 0:
        row = jax.lax.broadcasted_iota(jnp.int32, (TQ, kw), 0) + i * TQ
        col = jax.lax.broadcasted_iota(jnp.int32, (TQ, kw), 1)
        causal = col <= row
    onescol = (jax.lax.broadcasted_iota(jnp.int32, (kw, DH), 1) == 0
               ).astype(BF16)
    gsig = jax.nn.sigmoid(g_ref[...])
    outs = []
    for hh in range(2):
        lo, hi = hh * DH, (hh + 1) * DH
        q = q2[:, lo:hi]
        k = k2[:, lo:hi]
        v = v2[:, lo:hi]
        s = _dot_trhs(q, k) * SCALE
        c = jnp.max(s, axis=-1, keepdims=True)
        kc = _dot(_dot(M, k), wck_ref[...])
        vc = _dot(_dot(M, v), wcv_ref[...])
        scT = _dot_trhs(kc, q) * SCALE
        scmT = jnp.where(cmaskT, scT, NEG)
        mT = jnp.max(scmT, axis=0, keepdims=True)
        eT = jnp.exp(scmT - mT)
        pcT = eT / jnp.sum(eT, axis=0, keepdims=True)
        pcT = jnp.where(posT[:1] >= CBS - 1, pcT, 0.0)
        o_cmp = _dot_tlhs(pcT, vc)
        impT = jnp.where(cmaskT, pcT, -1.0)
        selT = jnp.zeros((nbk, TQ), jnp.bool_)
        for _ in range(NSEL):
            mx = jnp.max(impT, axis=0, keepdims=True)
            ismax = impT == mx
            first = jnp.min(jnp.where(ismax, blkT, nbk), axis=0, keepdims=True)
            onehot = blkT == first
            selT = selT | onehot
            impT = jnp.where(onehot, -2.0, impT)
        msel = _dot_tlhs(selT.astype(BF16), Ex)
        es = jnp.exp(s - c) * msel
        if i0 == 0:
            es = jnp.where(causal, es, 0.0)
        es = es.astype(BF16)
        vv = jnp.concatenate([v, onescol], axis=1)
        oz = jnp.dot(es, vv, preferred_element_type=F32)
        o_sel = oz[:, :DH] / oz[:, DH:DH + 1]
        kslab = kslab2[:, lo:hi]
        vslab = vslab2[:, lo:hi]
        sslab = _dot_trhs(q, kslab) * SCALE
        ew = jnp.where(wm, jnp.exp(sslab - c), 0.0).astype(BF16)
        vvs = jnp.concatenate([vslab, onescol[:2 * TQ]], axis=1)
        wz = jnp.dot(ew, vvs, preferred_element_type=F32)
        o_win = wz[:, :DH] / wz[:, DH:DH + 1]
        g = gsig[hh]
        outs.append((g[:, 0:1] * o_cmp + g[:, 1:2] * o_sel
                     + g[:, 2:3] * o_win).astype(BF16))
    o_ref[...] = jnp.concatenate(outs, axis=1)


def _attention_part(q, k, v, gates_h, Wck, Wcv, kw, i0, nt, nbk):
    body = functools.partial(_k3, kw=kw, i0=i0, nbk=nbk)
    return pl.pallas_call(
        body,
        grid=(HP, nt),
        in_specs=[
            pl.BlockSpec((TQ, 2 * DH), lambda h, i: (i + i0, h)),
            pl.BlockSpec((kw, 2 * DH), lambda h, i: (0, h)),
            pl.BlockSpec((kw, 2 * DH), lambda h, i: (0, h)),
            pl.BlockSpec((2, TQ, 3), lambda h, i: (h, i + i0, 0)),
            pl.BlockSpec((DH, DH), lambda h, i: (0, 0)),
            pl.BlockSpec((DH, DH), lambda h, i: (0, 0)),
        ],
        out_specs=pl.BlockSpec((TQ, 2 * DH), lambda h, i: (i, h)),
        out_shape=jax.ShapeDtypeStruct((nt * TQ, D), BF16),
    )(q, k, v, gates_h, Wck, Wcv)


def _k45(o_ref, x_ref, wo_ref, g_ref, b_ref, w1_ref, b1_ref, w2_ref, b2_ref,
         y_ref):
    x1 = x_ref[...] + _dot(o_ref[...], wo_ref[...])
    xn = _ln_body(x1, g_ref[...], b_ref[...])
    hgelu = jax.nn.gelu(_dot(xn, w1_ref[...]) + b1_ref[...])
    y_ref[...] = x1 + _dot(hgelu, w2_ref[...]) + b2_ref[...]


def _tail(o, x, Wo, ln_g, ln_b, W1, b1, W2, b2):
    return pl.pallas_call(
        _k45,
        grid=(NQT,),
        in_specs=[
            pl.BlockSpec((TQ, D), lambda i: (i, 0)),
            pl.BlockSpec((TQ, D), lambda i: (i, 0)),
            pl.BlockSpec((D, D), lambda i: (0, 0)),
            pl.BlockSpec((1, D), lambda i: (0, 0)),
            pl.BlockSpec((1, D), lambda i: (0, 0)),
            pl.BlockSpec((D, MLP), lambda i: (0, 0)),
            pl.BlockSpec((1, MLP), lambda i: (0, 0)),
            pl.BlockSpec((MLP, D), lambda i: (0, 0)),
            pl.BlockSpec((1, D), lambda i: (0, 0)),
        ],
        out_specs=pl.BlockSpec((TQ, D), lambda i: (i, 0)),
        out_shape=jax.ShapeDtypeStruct((S, D), F32),
    )(o, x, Wo, ln_g, ln_b, W1, b1, W2, b2)


@jax.jit
def _run(x, ln1_g, ln1_b, Wq, Wk, Wv, Wck, Wcv, Wg, Wo, ln2_g, ln2_b, W1, b1, W2, b2):
    x2 = x[0]
    q, k, v, gates = _proj(x2, ln1_g[None], ln1_b[None], Wq, Wk, Wv, Wg)
    gates_h = gates.reshape(S, H, 3).transpose(1, 0, 2)
    oA = _attention_part(q, k, v, gates_h, Wck, Wcv, S // 2, 0, NQT // 2,
                         NB // 2)
    oB = _attention_part(q, k, v, gates_h, Wck, Wcv, S, NQT // 2, NQT // 2,
                         NB)
    o = jnp.concatenate([oA, oB], axis=0)
    y = _tail(o, x2, Wo, ln2_g[None], ln2_b[None], W1, b1[None], W2, b2[None])
    return y[None]


def kernel(x, ln1_g, ln1_b, Wq, Wk, Wv, Wck, Wcv, Wg, Wo, ln2_g, ln2_b, W1, b1, W2, b2):
    return _run(x, ln1_g, ln1_b, Wq, Wk, Wv, Wck, Wcv, Wg, Wo,
                ln2_g, ln2_b, W1, b1, W2, b2)


# speedup vs baseline: 21.1135x; 1.2644x over previous
import functools

import jax
import jax.numpy as jnp
from jax.experimental import pallas as pl

B, S, D = 1, 2048, 768
H, DH = 12, 64
CBS = 32
SBS = 32
NSEL = 4
SW = 128
MLP = 3072
NB = S // CBS
SCALE = DH ** -0.5
NEG = -1e30

TQ = 256
NQT = S // TQ
ATQ = 512
HP = H // 2
WS = ATQ + 256

F32 = jnp.float32
BF16 = jnp.bfloat16


def _ln_body(xt, g, b):
    mu = jnp.mean(xt, axis=-1, keepdims=True)
    xc = xt - mu
    var = jnp.mean(xc * xc, axis=-1, keepdims=True)
    return xc * jax.lax.rsqrt(var + 1e-5) * g + b


def _dot(a, b):
    return jnp.dot(a.astype(BF16), b.astype(BF16), preferred_element_type=F32)


def _dot_tlhs(a, b, prefer=F32):
    return jax.lax.dot_general(a.astype(BF16), b.astype(BF16),
                               (((0,), (0,)), ((), ())),
                               preferred_element_type=prefer)


def _dot_trhs(a, b):
    return jax.lax.dot_general(a.astype(BF16), b.astype(BF16),
                               (((1,), (1,)), ((), ())),
                               preferred_element_type=F32)


def _k1(x_ref, g_ref, b_ref, wq_ref, wk_ref, wv_ref, wg_ref,
        q_ref, k_ref, v_ref, gates_ref):
    xn = _ln_body(x_ref[...], g_ref[...], b_ref[...]).astype(BF16)
    q_ref[...] = jnp.dot(xn, wq_ref[...],
                         preferred_element_type=F32).astype(BF16)
    k_ref[...] = jnp.dot(xn, wk_ref[...],
                         preferred_element_type=F32).astype(BF16)
    v_ref[...] = jnp.dot(xn, wv_ref[...],
                         preferred_element_type=F32).astype(BF16)
    gates_ref[...] = jnp.dot(xn, wg_ref[...], preferred_element_type=F32)


def _proj(x, ln_g, ln_b, Wq, Wk, Wv, Wg):
    return pl.pallas_call(
        _k1,
        grid=(NQT,),
        in_specs=[
            pl.BlockSpec((TQ, D), lambda i: (i, 0)),
            pl.BlockSpec((1, D), lambda i: (0, 0)),
            pl.BlockSpec((1, D), lambda i: (0, 0)),
            pl.BlockSpec((D, D), lambda i: (0, 0)),
            pl.BlockSpec((D, D), lambda i: (0, 0)),
            pl.BlockSpec((D, D), lambda i: (0, 0)),
            pl.BlockSpec((D, 3 * H), lambda i: (0, 0)),
        ],
        out_specs=[
            pl.BlockSpec((TQ, D), lambda i: (i, 0)),
            pl.BlockSpec((TQ, D), lambda i: (i, 0)),
            pl.BlockSpec((TQ, D), lambda i: (i, 0)),
            pl.BlockSpec((TQ, 3 * H), lambda i: (i, 0)),
        ],
        out_shape=[
            jax.ShapeDtypeStruct((S, D), BF16),
            jax.ShapeDtypeStruct((S, D), BF16),
            jax.ShapeDtypeStruct((S, D), BF16),
            jax.ShapeDtypeStruct((S, 3 * H), F32),
        ],
    )(x, ln_g, ln_b, Wq, Wk, Wv, Wg)


def _k3(q_ref, k_ref, v_ref, g_ref, wck_ref, wcv_ref, o_ref, *, ti, kw, nbk):
    row0 = ti * ATQ
    q2 = q_ref[...]
    k2 = k_ref[...]
    v2 = v_ref[...]
    n_i = jax.lax.broadcasted_iota(jnp.int32, (nbk, kw), 0)
    s_i = jax.lax.broadcasted_iota(jnp.int32, (nbk, kw), 1)
    Ex = jnp.where(s_i // CBS == n_i, 1.0, 0.0).astype(BF16)
    posT = jax.lax.broadcasted_iota(jnp.int32, (nbk, ATQ), 1) + row0
    blkT = jax.lax.broadcasted_iota(jnp.int32, (nbk, ATQ), 0)
    cmaskT = (blkT + 1) * CBS - 1 <= posT
    if ti == 0:
        row = jax.lax.broadcasted_iota(jnp.int32, (ATQ, kw), 0)
        col = jax.lax.broadcasted_iota(jnp.int32, (ATQ, kw), 1)
        causal = col <= row
        wmask = causal & (col > row - SW)
    else:
        colw = jax.lax.broadcasted_iota(jnp.int32, (ATQ, WS), 1) + row0 - 256
        roww = jax.lax.broadcasted_iota(jnp.int32, (ATQ, WS), 0) + row0
        wmask = (colw <= roww) & (colw > roww - SW)
    onescol = (jax.lax.broadcasted_iota(jnp.int32, (kw, DH), 1) == 0
               ).astype(BF16)
    gsig = jax.nn.sigmoid(g_ref[...])
    outs = []
    for hh in range(2):
        lo, hi = hh * DH, (hh + 1) * DH
        q = q2[:, lo:hi] * jnp.asarray(SCALE, BF16)
        k = k2[:, lo:hi]
        v = v2[:, lo:hi]
        s = _dot_trhs(q, k)
        c = jnp.max(s, axis=-1, keepdims=True)
        e = jnp.exp(s - c)
        kc = _dot(_dot(Ex, k) * (1.0 / CBS), wck_ref[...])
        vc = _dot(_dot(Ex, v) * (1.0 / CBS), wcv_ref[...])
        scT = _dot_trhs(kc, q)
        scmT = jnp.where(cmaskT, scT, NEG)
        mT = jnp.max(scmT, axis=0, keepdims=True)
        eT = jnp.exp(scmT - mT)
        pcT = eT / jnp.sum(eT, axis=0, keepdims=True)
        pcT = jnp.where(posT[:1] >= CBS - 1, pcT, 0.0)
        o_cmp = _dot_tlhs(pcT, vc)
        impT = jnp.where(cmaskT, pcT, -1.0)
        selT = jnp.zeros((nbk, ATQ), jnp.bool_)
        for _ in range(NSEL):
            mx = jnp.max(impT, axis=0, keepdims=True)
            ismax = impT == mx
            first = jnp.min(jnp.where(ismax, blkT, nbk), axis=0, keepdims=True)
            onehot = blkT == first
            selT = selT | onehot
            impT = jnp.where(onehot, -2.0, impT)
        msel = _dot_tlhs(selT.astype(BF16), Ex)
        es = e * msel
        if ti == 0:
            es = jnp.where(causal, es, 0.0)
        es = es.astype(BF16)
        vv = jnp.concatenate([v, onescol], axis=1)
        oz = jnp.dot(es, vv, preferred_element_type=F32)
        o_sel = oz[:, :DH] / oz[:, DH:DH + 1]
        if ti == 0:
            ew = jnp.where(wmask, e, 0.0).astype(BF16)
            wz = jnp.dot(ew, vv, preferred_element_type=F32)
        else:
            kslab = k[row0 - 256:row0 + ATQ]
            vslab = v[row0 - 256:row0 + ATQ]
            sslab = _dot_trhs(q, kslab)
            ew = jnp.where(wmask, jnp.exp(sslab - c), 0.0).astype(BF16)
            vvs = jnp.concatenate([vslab, onescol[:WS]], axis=1)
            wz = jnp.dot(ew, vvs, preferred_element_type=F32)
        o_win = wz[:, :DH] / wz[:, DH:DH + 1]
        g = gsig[hh]
        outs.append((g[:, 0:1] * o_cmp + g[:, 1:2] * o_sel
                     + g[:, 2:3] * o_win).astype(BF16))
    o_ref[...] = jnp.concatenate(outs, axis=1)


def _attention_tile(q, k, v, gates_h, Wck, Wcv, ti):
    kw = (ti + 1) * ATQ
    nbk = kw // CBS
    body = functools.partial(_k3, ti=ti, kw=kw, nbk=nbk)
    return pl.pallas_call(
        body,
        grid=(HP,),
        in_specs=[
            pl.BlockSpec((ATQ, 2 * DH), lambda h: (ti, h)),
            pl.BlockSpec((kw, 2 * DH), lambda h: (0, h)),
            pl.BlockSpec((kw, 2 * DH), lambda h: (0, h)),
            pl.BlockSpec((2, ATQ, 3), lambda h: (h, ti, 0)),
            pl.BlockSpec((DH, DH), lambda h: (0, 0)),
            pl.BlockSpec((DH, DH), lambda h: (0, 0)),
        ],
        out_specs=pl.BlockSpec((ATQ, 2 * DH), lambda h: (0, h)),
        out_shape=jax.ShapeDtypeStruct((ATQ, D), BF16),
    )(q, k, v, gates_h, Wck, Wcv)


def _k45(o_ref, x_ref, wo_ref, g_ref, b_ref, w1_ref, b1_ref, w2_ref, b2_ref,
         y_ref):
    x1 = x_ref[...] + jnp.dot(o_ref[...], wo_ref[...],
                              preferred_element_type=F32)
    xn = _ln_body(x1, g_ref[...], b_ref[...]).astype(BF16)
    hgelu = jax.nn.gelu(jnp.dot(xn, w1_ref[...], preferred_element_type=F32)
                        + b1_ref[...])
    y_ref[...] = x1 + jnp.dot(hgelu.astype(BF16), w2_ref[...],
                              preferred_element_type=F32) + b2_ref[...]


def _tail(o, x, Wo, ln_g, ln_b, W1, b1, W2, b2):
    return pl.pallas_call(
        _k45,
        grid=(NQT,),
        in_specs=[
            pl.BlockSpec((TQ, D), lambda i: (i, 0)),
            pl.BlockSpec((TQ, D), lambda i: (i, 0)),
            pl.BlockSpec((D, D), lambda i: (0, 0)),
            pl.BlockSpec((1, D), lambda i: (0, 0)),
            pl.BlockSpec((1, D), lambda i: (0, 0)),
            pl.BlockSpec((D, MLP), lambda i: (0, 0)),
            pl.BlockSpec((1, MLP), lambda i: (0, 0)),
            pl.BlockSpec((MLP, D), lambda i: (0, 0)),
            pl.BlockSpec((1, D), lambda i: (0, 0)),
        ],
        out_specs=pl.BlockSpec((TQ, D), lambda i: (i, 0)),
        out_shape=jax.ShapeDtypeStruct((S, D), F32),
    )(o, x, Wo, ln_g, ln_b, W1, b1, W2, b2)


@jax.jit
def _run(x, ln1_g, ln1_b, Wq, Wk, Wv, Wck, Wcv, Wg, Wo, ln2_g, ln2_b, W1, b1, W2, b2):
    x2 = x[0]
    q, k, v, gates = _proj(x2, ln1_g[None], ln1_b[None],
                           Wq.astype(BF16), Wk.astype(BF16), Wv.astype(BF16),
                           Wg.astype(BF16))
    gates_h = gates.reshape(S, H, 3).transpose(1, 0, 2)
    Wckb = Wck.astype(BF16)
    Wcvb = Wcv.astype(BF16)
    o = jnp.concatenate(
        [_attention_tile(q, k, v, gates_h, Wckb, Wcvb, ti) for ti in range(4)],
        axis=0)
    y = _tail(o, x2, Wo.astype(BF16), ln2_g[None], ln2_b[None],
              W1.astype(BF16), b1[None], W2.astype(BF16), b2[None])
    return y[None]


def kernel(x, ln1_g, ln1_b, Wq, Wk, Wv, Wck, Wcv, Wg, Wo, ln2_g, ln2_b, W1, b1, W2, b2):
    return _run(x, ln1_g, ln1_b, Wq, Wk, Wv, Wck, Wcv, Wg, Wo,
                ln2_g, ln2_b, W1, b1, W2, b2)


# speedup vs baseline: 21.4894x; 1.0178x over previous
import functools

import jax
import jax.numpy as jnp
from jax.experimental import pallas as pl

B, S, D = 1, 2048, 768
H, DH = 12, 64
CBS = 32
SBS = 32
NSEL = 4
SW = 128
MLP = 3072
NB = S // CBS
SCALE = DH ** -0.5
NEG = -1e30

TQ = 512
NQT = S // TQ
ATQ = 512
HP = H // 2
WS = ATQ + 256

F32 = jnp.float32
BF16 = jnp.bfloat16


def _ln_body(xt, g, b):
    mu = jnp.mean(xt, axis=-1, keepdims=True)
    xc = xt - mu
    var = jnp.mean(xc * xc, axis=-1, keepdims=True)
    return xc * jax.lax.rsqrt(var + 1e-5) * g + b


def _dot(a, b):
    return jnp.dot(a.astype(BF16), b.astype(BF16), preferred_element_type=F32)


def _dot_tlhs(a, b, prefer=F32):
    return jax.lax.dot_general(a.astype(BF16), b.astype(BF16),
                               (((0,), (0,)), ((), ())),
                               preferred_element_type=prefer)


def _dot_trhs(a, b):
    return jax.lax.dot_general(a.astype(BF16), b.astype(BF16),
                               (((1,), (1,)), ((), ())),
                               preferred_element_type=F32)


def _k1(x_ref, g_ref, b_ref, wq_ref, wk_ref, wv_ref, wg_ref,
        q_ref, k_ref, v_ref, gates_ref):
    xn = _ln_body(x_ref[...], g_ref[...], b_ref[...]).astype(BF16)
    q_ref[...] = jnp.dot(xn, wq_ref[...],
                         preferred_element_type=F32).astype(BF16)
    k_ref[...] = jnp.dot(xn, wk_ref[...],
                         preferred_element_type=F32).astype(BF16)
    v_ref[...] = jnp.dot(xn, wv_ref[...],
                         preferred_element_type=F32).astype(BF16)
    gates_ref[...] = jnp.dot(xn, wg_ref[...], preferred_element_type=F32)


def _proj(x, ln_g, ln_b, Wq, Wk, Wv, Wg):
    return pl.pallas_call(
        _k1,
        grid=(NQT,),
        in_specs=[
            pl.BlockSpec((TQ, D), lambda i: (i, 0)),
            pl.BlockSpec((1, D), lambda i: (0, 0)),
            pl.BlockSpec((1, D), lambda i: (0, 0)),
            pl.BlockSpec((D, D), lambda i: (0, 0)),
            pl.BlockSpec((D, D), lambda i: (0, 0)),
            pl.BlockSpec((D, D), lambda i: (0, 0)),
            pl.BlockSpec((D, 3 * H), lambda i: (0, 0)),
        ],
        out_specs=[
            pl.BlockSpec((TQ, D), lambda i: (i, 0)),
            pl.BlockSpec((TQ, D), lambda i: (i, 0)),
            pl.BlockSpec((TQ, D), lambda i: (i, 0)),
            pl.BlockSpec((TQ, 3 * H), lambda i: (i, 0)),
        ],
        out_shape=[
            jax.ShapeDtypeStruct((S, D), BF16),
            jax.ShapeDtypeStruct((S, D), BF16),
            jax.ShapeDtypeStruct((S, D), BF16),
            jax.ShapeDtypeStruct((S, 3 * H), F32),
        ],
    )(x, ln_g, ln_b, Wq, Wk, Wv, Wg)


def _k3(q_ref, k_ref, v_ref, g_ref, wck_ref, wcv_ref, o_ref, *, ti, kw, nbk):
    row0 = ti * ATQ
    q2 = q_ref[...]
    k2 = k_ref[...]
    v2 = v_ref[...]
    n_i = jax.lax.broadcasted_iota(jnp.int32, (nbk, kw), 0)
    s_i = jax.lax.broadcasted_iota(jnp.int32, (nbk, kw), 1)
    Ex = jnp.where(s_i // CBS == n_i, 1.0, 0.0).astype(BF16)
    posT = jax.lax.broadcasted_iota(jnp.int32, (nbk, ATQ), 1) + row0
    blkT = jax.lax.broadcasted_iota(jnp.int32, (nbk, ATQ), 0)
    cmaskT = (blkT + 1) * CBS - 1 <= posT
    if ti == 0:
        row = jax.lax.broadcasted_iota(jnp.int32, (ATQ, kw), 0)
        col = jax.lax.broadcasted_iota(jnp.int32, (ATQ, kw), 1)
        causal = col <= row
        wmask = causal & (col > row - SW)
    else:
        colw = jax.lax.broadcasted_iota(jnp.int32, (ATQ, WS), 1) + row0 - 256
        roww = jax.lax.broadcasted_iota(jnp.int32, (ATQ, WS), 0) + row0
        wmask = (colw <= roww) & (colw > roww - SW)
    onescol = (jax.lax.broadcasted_iota(jnp.int32, (kw, DH), 1) == 0
               ).astype(BF16)
    gsig = jax.nn.sigmoid(g_ref[...])
    outs = []
    for hh in range(2):
        lo, hi = hh * DH, (hh + 1) * DH
        q = q2[:, lo:hi] * jnp.asarray(SCALE, BF16)
        k = k2[:, lo:hi]
        v = v2[:, lo:hi]
        s = _dot_trhs(q, k)
        c = jnp.max(s, axis=-1, keepdims=True)
        e = jnp.exp(s - c)
        kc = _dot(_dot(Ex, k) * (1.0 / CBS), wck_ref[...])
        vc = _dot(_dot(Ex, v) * (1.0 / CBS), wcv_ref[...])
        scT = _dot_trhs(kc, q)
        scmT = jnp.where(cmaskT, scT, NEG)
        mT = jnp.max(scmT, axis=0, keepdims=True)
        eT = jnp.exp(scmT - mT)
        pcT = eT / jnp.sum(eT, axis=0, keepdims=True)
        pcT = jnp.where(posT[:1] >= CBS - 1, pcT, 0.0)
        o_cmp = _dot_tlhs(pcT, vc)
        impT = jnp.where(cmaskT, pcT, -1.0)
        selT = jnp.zeros((nbk, ATQ), jnp.bool_)
        for _ in range(NSEL):
            mx = jnp.max(impT, axis=0, keepdims=True)
            ismax = impT == mx
            first = jnp.min(jnp.where(ismax, blkT, nbk), axis=0, keepdims=True)
            onehot = blkT == first
            selT = selT | onehot
            impT = jnp.where(onehot, -2.0, impT)
        msel = _dot_tlhs(selT.astype(BF16), Ex)
        es = e * msel
        if ti == 0:
            es = jnp.where(causal, es, 0.0)
        es = es.astype(BF16)
        vv = jnp.concatenate([v, onescol], axis=1)
        oz = jnp.dot(es, vv, preferred_element_type=F32)
        o_sel = oz[:, :DH] / oz[:, DH:DH + 1]
        if ti == 0:
            ew = jnp.where(wmask, e, 0.0).astype(BF16)
            wz = jnp.dot(ew, vv, preferred_element_type=F32)
        else:
            kslab = k[row0 - 256:row0 + ATQ]
            vslab = v[row0 - 256:row0 + ATQ]
            sslab = _dot_trhs(q, kslab)
            ew = jnp.where(wmask, jnp.exp(sslab - c), 0.0).astype(BF16)
            vvs = jnp.concatenate([vslab, onescol[:WS]], axis=1)
            wz = jnp.dot(ew, vvs, preferred_element_type=F32)
        o_win = wz[:, :DH] / wz[:, DH:DH + 1]
        g = gsig[hh]
        outs.append((g[:, 0:1] * o_cmp + g[:, 1:2] * o_sel
                     + g[:, 2:3] * o_win).astype(BF16))
    o_ref[...] = jnp.concatenate(outs, axis=1)


def _attention_tile(q, k, v, gates_h, Wck, Wcv, ti):
    kw = (ti + 1) * ATQ
    nbk = kw // CBS
    body = functools.partial(_k3, ti=ti, kw=kw, nbk=nbk)
    return pl.pallas_call(
        body,
        grid=(HP,),
        in_specs=[
            pl.BlockSpec((ATQ, 2 * DH), lambda h: (ti, h)),
            pl.BlockSpec((kw, 2 * DH), lambda h: (0, h)),
            pl.BlockSpec((kw, 2 * DH), lambda h: (0, h)),
            pl.BlockSpec((2, ATQ, 3), lambda h: (h, ti, 0)),
            pl.BlockSpec((DH, DH), lambda h: (0, 0)),
            pl.BlockSpec((DH, DH), lambda h: (0, 0)),
        ],
        out_specs=pl.BlockSpec((ATQ, 2 * DH), lambda h: (0, h)),
        out_shape=jax.ShapeDtypeStruct((ATQ, D), BF16),
    )(q, k, v, gates_h, Wck, Wcv)


def _k45(o_ref, x_ref, wo_ref, g_ref, b_ref, w1_ref, b1_ref, w2_ref, b2_ref,
         y_ref):
    x1 = x_ref[...] + jnp.dot(o_ref[...], wo_ref[...],
                              preferred_element_type=F32)
    xn = _ln_body(x1, g_ref[...], b_ref[...]).astype(BF16)
    hgelu = jax.nn.gelu(jnp.dot(xn, w1_ref[...], preferred_element_type=F32)
                        + b1_ref[...])
    y_ref[...] = x1 + jnp.dot(hgelu.astype(BF16), w2_ref[...],
                              preferred_element_type=F32) + b2_ref[...]


def _tail(o, x, Wo, ln_g, ln_b, W1, b1, W2, b2):
    return pl.pallas_call(
        _k45,
        grid=(NQT,),
        in_specs=[
            pl.BlockSpec((TQ, D), lambda i: (i, 0)),
            pl.BlockSpec((TQ, D), lambda i: (i, 0)),
            pl.BlockSpec((D, D), lambda i: (0, 0)),
            pl.BlockSpec((1, D), lambda i: (0, 0)),
            pl.BlockSpec((1, D), lambda i: (0, 0)),
            pl.BlockSpec((D, MLP), lambda i: (0, 0)),
            pl.BlockSpec((1, MLP), lambda i: (0, 0)),
            pl.BlockSpec((MLP, D), lambda i: (0, 0)),
            pl.BlockSpec((1, D), lambda i: (0, 0)),
        ],
        out_specs=pl.BlockSpec((TQ, D), lambda i: (i, 0)),
        out_shape=jax.ShapeDtypeStruct((S, D), F32),
    )(o, x, Wo, ln_g, ln_b, W1, b1, W2, b2)


@jax.jit
def _run(x, ln1_g, ln1_b, Wq, Wk, Wv, Wck, Wcv, Wg, Wo, ln2_g, ln2_b, W1, b1, W2, b2):
    x2 = x[0]
    q, k, v, gates = _proj(x2, ln1_g[None], ln1_b[None],
                           Wq.astype(BF16), Wk.astype(BF16), Wv.astype(BF16),
                           Wg.astype(BF16))
    gates_h = gates.reshape(S, H, 3).transpose(1, 0, 2)
    Wckb = Wck.astype(BF16)
    Wcvb = Wcv.astype(BF16)
    o = jnp.concatenate(
        [_attention_tile(q, k, v, gates_h, Wckb, Wcvb, ti) for ti in range(4)],
        axis=0)
    y = _tail(o, x2, Wo.astype(BF16), ln2_g[None], ln2_b[None],
              W1.astype(BF16), b1[None], W2.astype(BF16), b2[None])
    return y[None]


def kernel(x, ln1_g, ln1_b, Wq, Wk, Wv, Wck, Wcv, Wg, Wo, ln2_g, ln2_b, W1, b1, W2, b2):
    return _run(x, ln1_g, ln1_b, Wq, Wk, Wv, Wck, Wcv, Wg, Wo,
                ln2_g, ln2_b, W1, b1, W2, b2)


# speedup vs baseline: 21.6330x; 1.0067x over previous
import functools

import jax
import jax.numpy as jnp
from jax.experimental import pallas as pl

B, S, D = 1, 2048, 768
H, DH = 12, 64
CBS = 32
SBS = 32
NSEL = 4
SW = 128
MLP = 3072
NB = S // CBS
SCALE = DH ** -0.5
NEG = -1e30

TQ = 512
NQT = S // TQ
ATQ = 512
HG = 4
HP = H // HG
WS = ATQ + 256

F32 = jnp.float32
BF16 = jnp.bfloat16


def _ln_body(xt, g, b):
    mu = jnp.mean(xt, axis=-1, keepdims=True)
    xc = xt - mu
    var = jnp.mean(xc * xc, axis=-1, keepdims=True)
    return xc * jax.lax.rsqrt(var + 1e-5) * g + b


def _dot(a, b):
    return jnp.dot(a.astype(BF16), b.astype(BF16), preferred_element_type=F32)


def _dot_tlhs(a, b, prefer=F32):
    return jax.lax.dot_general(a.astype(BF16), b.astype(BF16),
                               (((0,), (0,)), ((), ())),
                               preferred_element_type=prefer)


def _dot_trhs(a, b):
    return jax.lax.dot_general(a.astype(BF16), b.astype(BF16),
                               (((1,), (1,)), ((), ())),
                               preferred_element_type=F32)


def _k1(x_ref, g_ref, b_ref, wq_ref, wk_ref, wv_ref, wg_ref,
        q_ref, k_ref, v_ref, gates_ref):
    xn = _ln_body(x_ref[...], g_ref[...], b_ref[...]).astype(BF16)
    q_ref[...] = jnp.dot(xn, wq_ref[...],
                         preferred_element_type=F32).astype(BF16)
    k_ref[...] = jnp.dot(xn, wk_ref[...],
                         preferred_element_type=F32).astype(BF16)
    v_ref[...] = jnp.dot(xn, wv_ref[...],
                         preferred_element_type=F32).astype(BF16)
    gates_ref[...] = jnp.dot(xn, wg_ref[...], preferred_element_type=F32)


def _proj(x, ln_g, ln_b, Wq, Wk, Wv, Wg):
    return pl.pallas_call(
        _k1,
        grid=(NQT,),
        in_specs=[
            pl.BlockSpec((TQ, D), lambda i: (i, 0)),
            pl.BlockSpec((1, D), lambda i: (0, 0)),
            pl.BlockSpec((1, D), lambda i: (0, 0)),
            pl.BlockSpec((D, D), lambda i: (0, 0)),
            pl.BlockSpec((D, D), lambda i: (0, 0)),
            pl.BlockSpec((D, D), lambda i: (0, 0)),
            pl.BlockSpec((D, 3 * H), lambda i: (0, 0)),
        ],
        out_specs=[
            pl.BlockSpec((TQ, D), lambda i: (i, 0)),
            pl.BlockSpec((TQ, D), lambda i: (i, 0)),
            pl.BlockSpec((TQ, D), lambda i: (i, 0)),
            pl.BlockSpec((TQ, 3 * H), lambda i: (i, 0)),
        ],
        out_shape=[
            jax.ShapeDtypeStruct((S, D), BF16),
            jax.ShapeDtypeStruct((S, D), BF16),
            jax.ShapeDtypeStruct((S, D), BF16),
            jax.ShapeDtypeStruct((S, 3 * H), F32),
        ],
    )(x, ln_g, ln_b, Wq, Wk, Wv, Wg)


def _k3(q_ref, k_ref, v_ref, g_ref, wck_ref, wcv_ref, o_ref, *, ti, kw, nbk):
    row0 = ti * ATQ
    q2 = q_ref[...]
    k2 = k_ref[...]
    v2 = v_ref[...]
    n_i = jax.lax.broadcasted_iota(jnp.int32, (nbk, kw), 0)
    s_i = jax.lax.broadcasted_iota(jnp.int32, (nbk, kw), 1)
    Ex = jnp.where(s_i // CBS == n_i, 1.0, 0.0).astype(BF16)
    posT = jax.lax.broadcasted_iota(jnp.int32, (nbk, ATQ), 1) + row0
    blkT = jax.lax.broadcasted_iota(jnp.int32, (nbk, ATQ), 0)
    cmaskT = (blkT + 1) * CBS - 1 <= posT
    if ti == 0:
        row = jax.lax.broadcasted_iota(jnp.int32, (ATQ, kw), 0)
        col = jax.lax.broadcasted_iota(jnp.int32, (ATQ, kw), 1)
        causal = col <= row
        wmask = causal & (col > row - SW)
    else:
        colw = jax.lax.broadcasted_iota(jnp.int32, (ATQ, WS), 1) + row0 - 256
        roww = jax.lax.broadcasted_iota(jnp.int32, (ATQ, WS), 0) + row0
        wmask = (colw <= roww) & (colw > roww - SW)
    onescol = (jax.lax.broadcasted_iota(jnp.int32, (kw, DH), 1) == 0
               ).astype(BF16)
    gsig = jax.nn.sigmoid(g_ref[...])
    outs = []
    for hh in range(HG):
        lo, hi = hh * DH, (hh + 1) * DH
        q = q2[:, lo:hi] * jnp.asarray(SCALE, BF16)
        k = k2[:, lo:hi]
        v = v2[:, lo:hi]
        s = _dot_trhs(q, k)
        c = jnp.max(s, axis=-1, keepdims=True)
        e = jnp.exp(s - c)
        kc = _dot(_dot(Ex, k) * (1.0 / CBS), wck_ref[...])
        vc = _dot(_dot(Ex, v) * (1.0 / CBS), wcv_ref[...])
        scT = _dot_trhs(kc, q)
        scmT = jnp.where(cmaskT, scT, NEG)
        mT = jnp.max(scmT, axis=0, keepdims=True)
        eT = jnp.exp(scmT - mT)
        pcT = eT / jnp.sum(eT, axis=0, keepdims=True)
        pcT = jnp.where(posT[:1] >= CBS - 1, pcT, 0.0)
        o_cmp = _dot_tlhs(pcT, vc)
        impT = jnp.where(cmaskT, pcT, -1.0)
        selT = jnp.zeros((nbk, ATQ), jnp.bool_)
        for _ in range(NSEL):
            mx = jnp.max(impT, axis=0, keepdims=True)
            ismax = impT == mx
            first = jnp.min(jnp.where(ismax, blkT, nbk), axis=0, keepdims=True)
            onehot = blkT == first
            selT = selT | onehot
            impT = jnp.where(onehot, -2.0, impT)
        msel = _dot_tlhs(selT.astype(BF16), Ex)
        es = e * msel
        if ti == 0:
            es = jnp.where(causal, es, 0.0)
        es = es.astype(BF16)
        vv = jnp.concatenate([v, onescol], axis=1)
        oz = jnp.dot(es, vv, preferred_element_type=F32)
        o_sel = oz[:, :DH] / oz[:, DH:DH + 1]
        if ti == 0:
            ew = jnp.where(wmask, e, 0.0).astype(BF16)
            wz = jnp.dot(ew, vv, preferred_element_type=F32)
        else:
            kslab = k[row0 - 256:row0 + ATQ]
            vslab = v[row0 - 256:row0 + ATQ]
            sslab = _dot_trhs(q, kslab)
            ew = jnp.where(wmask, jnp.exp(sslab - c), 0.0).astype(BF16)
            vvs = jnp.concatenate([vslab, onescol[:WS]], axis=1)
            wz = jnp.dot(ew, vvs, preferred_element_type=F32)
        o_win = wz[:, :DH] / wz[:, DH:DH + 1]
        g = gsig[hh]
        outs.append((g[:, 0:1] * o_cmp + g[:, 1:2] * o_sel
                     + g[:, 2:3] * o_win).astype(BF16))
    o_ref[...] = jnp.concatenate(outs, axis=1)


def _attention_tile(q, k, v, gates_h, Wck, Wcv, ti):
    kw = (ti + 1) * ATQ
    nbk = kw // CBS
    body = functools.partial(_k3, ti=ti, kw=kw, nbk=nbk)
    return pl.pallas_call(
        body,
        grid=(HP,),
        in_specs=[
            pl.BlockSpec((ATQ, HG * DH), lambda h: (ti, h)),
            pl.BlockSpec((kw, HG * DH), lambda h: (0, h)),
            pl.BlockSpec((kw, HG * DH), lambda h: (0, h)),
            pl.BlockSpec((HG, ATQ, 3), lambda h: (h, ti, 0)),
            pl.BlockSpec((DH, DH), lambda h: (0, 0)),
            pl.BlockSpec((DH, DH), lambda h: (0, 0)),
        ],
        out_specs=pl.BlockSpec((ATQ, HG * DH), lambda h: (0, h)),
        out_shape=jax.ShapeDtypeStruct((ATQ, D), BF16),
    )(q, k, v, gates_h, Wck, Wcv)


def _k45(o_ref, x_ref, wo_ref, g_ref, b_ref, w1_ref, b1_ref, w2_ref, b2_ref,
         y_ref):
    x1 = x_ref[...] + jnp.dot(o_ref[...], wo_ref[...],
                              preferred_element_type=F32)
    xn = _ln_body(x1, g_ref[...], b_ref[...]).astype(BF16)
    hgelu = jax.nn.gelu(jnp.dot(xn, w1_ref[...], preferred_element_type=F32)
                        + b1_ref[...])
    y_ref[...] = x1 + jnp.dot(hgelu.astype(BF16), w2_ref[...],
                              preferred_element_type=F32) + b2_ref[...]


def _tail(o, x, Wo, ln_g, ln_b, W1, b1, W2, b2):
    return pl.pallas_call(
        _k45,
        grid=(NQT,),
        in_specs=[
            pl.BlockSpec((TQ, D), lambda i: (i, 0)),
            pl.BlockSpec((TQ, D), lambda i: (i, 0)),
            pl.BlockSpec((D, D), lambda i: (0, 0)),
            pl.BlockSpec((1, D), lambda i: (0, 0)),
            pl.BlockSpec((1, D), lambda i: (0, 0)),
            pl.BlockSpec((D, MLP), lambda i: (0, 0)),
            pl.BlockSpec((1, MLP), lambda i: (0, 0)),
            pl.BlockSpec((MLP, D), lambda i: (0, 0)),
            pl.BlockSpec((1, D), lambda i: (0, 0)),
        ],
        out_specs=pl.BlockSpec((TQ, D), lambda i: (i, 0)),
        out_shape=jax.ShapeDtypeStruct((S, D), F32),
    )(o, x, Wo, ln_g, ln_b, W1, b1, W2, b2)


@jax.jit
def _run(x, ln1_g, ln1_b, Wq, Wk, Wv, Wck, Wcv, Wg, Wo, ln2_g, ln2_b, W1, b1, W2, b2):
    x2 = x[0]
    q, k, v, gates = _proj(x2, ln1_g[None], ln1_b[None],
                           Wq.astype(BF16), Wk.astype(BF16), Wv.astype(BF16),
                           Wg.astype(BF16))
    gates_h = gates.reshape(S, H, 3).transpose(1, 0, 2)
    Wckb = Wck.astype(BF16)
    Wcvb = Wcv.astype(BF16)
    o = jnp.concatenate(
        [_attention_tile(q, k, v, gates_h, Wckb, Wcvb, ti) for ti in range(4)],
        axis=0)
    y = _tail(o, x2, Wo.astype(BF16), ln2_g[None], ln2_b[None],
              W1.astype(BF16), b1[None], W2.astype(BF16), b2[None])
    return y[None]


def kernel(x, ln1_g, ln1_b, Wq, Wk, Wv, Wck, Wcv, Wg, Wo, ln2_g, ln2_b, W1, b1, W2, b2):
    return _run(x, ln1_g, ln1_b, Wq, Wk, Wv, Wck, Wcv, Wg, Wo,
                ln2_g, ln2_b, W1, b1, W2, b2)


# speedup vs baseline: 22.6605x; 1.0475x over previous
import functools

import jax
import jax.numpy as jnp
from jax.experimental import pallas as pl
from jax.experimental.pallas import tpu as pltpu

B, S, D = 1, 2048, 768
H, DH = 12, 64
CBS = 32
SBS = 32
NSEL = 4
SW = 128
MLP = 3072
NB = S // CBS
SCALE = DH ** -0.5
NEG = -1e30

TQ = 512
NQT = S // TQ
ATQ = 512
HG = 4
HP = H // HG
WS = ATQ + 256

F32 = jnp.float32
BF16 = jnp.bfloat16


def _ln_body(xt, g, b):
    mu = jnp.mean(xt, axis=-1, keepdims=True)
    xc = xt - mu
    var = jnp.mean(xc * xc, axis=-1, keepdims=True)
    return xc * jax.lax.rsqrt(var + 1e-5) * g + b


def _dot(a, b):
    return jnp.dot(a.astype(BF16), b.astype(BF16), preferred_element_type=F32)


def _dot_tlhs(a, b, prefer=F32):
    return jax.lax.dot_general(a.astype(BF16), b.astype(BF16),
                               (((0,), (0,)), ((), ())),
                               preferred_element_type=prefer)


def _dot_trhs(a, b):
    return jax.lax.dot_general(a.astype(BF16), b.astype(BF16),
                               (((1,), (1,)), ((), ())),
                               preferred_element_type=F32)


def _k1(x_ref, g_ref, b_ref, wq_ref, wk_ref, wv_ref, wg_ref,
        q_ref, k_ref, v_ref, gates_ref, wqb_ref, wkb_ref, wvb_ref):
    @pl.when(pl.program_id(0) == 0)
    def _():
        wqb_ref[...] = wq_ref[...].astype(BF16)
        wkb_ref[...] = wk_ref[...].astype(BF16)
        wvb_ref[...] = wv_ref[...].astype(BF16)
    xn = _ln_body(x_ref[...], g_ref[...], b_ref[...]).astype(BF16)
    q_ref[...] = jnp.dot(xn, wqb_ref[...],
                         preferred_element_type=F32).astype(BF16)
    k_ref[...] = jnp.dot(xn, wkb_ref[...],
                         preferred_element_type=F32).astype(BF16)
    v_ref[...] = jnp.dot(xn, wvb_ref[...],
                         preferred_element_type=F32).astype(BF16)
    gates_ref[...] = _dot(xn, wg_ref[...])


def _proj(x, ln_g, ln_b, Wq, Wk, Wv, Wg):
    return pl.pallas_call(
        _k1,
        grid=(NQT,),
        in_specs=[
            pl.BlockSpec((TQ, D), lambda i: (i, 0)),
            pl.BlockSpec((1, D), lambda i: (0, 0)),
            pl.BlockSpec((1, D), lambda i: (0, 0)),
            pl.BlockSpec((D, D), lambda i: (0, 0)),
            pl.BlockSpec((D, D), lambda i: (0, 0)),
            pl.BlockSpec((D, D), lambda i: (0, 0)),
            pl.BlockSpec((D, 3 * H), lambda i: (0, 0)),
        ],
        out_specs=[
            pl.BlockSpec((TQ, D), lambda i: (i, 0)),
            pl.BlockSpec((TQ, D), lambda i: (i, 0)),
            pl.BlockSpec((TQ, D), lambda i: (i, 0)),
            pl.BlockSpec((TQ, 3 * H), lambda i: (i, 0)),
        ],
        out_shape=[
            jax.ShapeDtypeStruct((S, D), BF16),
            jax.ShapeDtypeStruct((S, D), BF16),
            jax.ShapeDtypeStruct((S, D), BF16),
            jax.ShapeDtypeStruct((S, 3 * H), F32),
        ],
        scratch_shapes=[
            pltpu.VMEM((D, D), BF16),
            pltpu.VMEM((D, D), BF16),
            pltpu.VMEM((D, D), BF16),
        ],
    )(x, ln_g, ln_b, Wq, Wk, Wv, Wg)


def _k3(q_ref, k_ref, v_ref, g_ref, wck_ref, wcv_ref, o_ref, *, ti, kw, nbk):
    row0 = ti * ATQ
    q2 = q_ref[...]
    k2 = k_ref[...]
    v2 = v_ref[...]
    n_i = jax.lax.broadcasted_iota(jnp.int32, (nbk, kw), 0)
    s_i = jax.lax.broadcasted_iota(jnp.int32, (nbk, kw), 1)
    Ex = jnp.where(s_i // CBS == n_i, 1.0, 0.0).astype(BF16)
    posT = jax.lax.broadcasted_iota(jnp.int32, (nbk, ATQ), 1) + row0
    blkT = jax.lax.broadcasted_iota(jnp.int32, (nbk, ATQ), 0)
    cmaskT = (blkT + 1) * CBS - 1 <= posT
    if ti == 0:
        row = jax.lax.broadcasted_iota(jnp.int32, (ATQ, kw), 0)
        col = jax.lax.broadcasted_iota(jnp.int32, (ATQ, kw), 1)
        causal = col <= row
        wmask = causal & (col > row - SW)
    else:
        colw = jax.lax.broadcasted_iota(jnp.int32, (ATQ, WS), 1) + row0 - 256
        roww = jax.lax.broadcasted_iota(jnp.int32, (ATQ, WS), 0) + row0
        wmask = (colw <= roww) & (colw > roww - SW)
    onescol = (jax.lax.broadcasted_iota(jnp.int32, (kw, DH), 1) == 0
               ).astype(BF16)
    gsig = jax.nn.sigmoid(g_ref[...])
    outs = []
    for hh in range(HG):
        lo, hi = hh * DH, (hh + 1) * DH
        q = q2[:, lo:hi] * jnp.asarray(SCALE, BF16)
        k = k2[:, lo:hi]
        v = v2[:, lo:hi]
        s = _dot_trhs(q, k)
        c = jnp.max(s, axis=-1, keepdims=True)
        e = jnp.exp(s - c)
        kc = _dot(_dot(Ex, k) * (1.0 / CBS), wck_ref[...])
        vc = _dot(_dot(Ex, v) * (1.0 / CBS), wcv_ref[...])
        scT = _dot_trhs(kc, q)
        scmT = jnp.where(cmaskT, scT, NEG)
        mT = jnp.max(scmT, axis=0, keepdims=True)
        eT = jnp.exp(scmT - mT)
        pcT = eT / jnp.sum(eT, axis=0, keepdims=True)
        pcT = jnp.where(posT[:1] >= CBS - 1, pcT, 0.0)
        o_cmp = _dot_tlhs(pcT, vc)
        impT = jnp.where(cmaskT, pcT, -1.0)
        selT = jnp.zeros((nbk, ATQ), jnp.bool_)
        for _ in range(NSEL):
            mx = jnp.max(impT, axis=0, keepdims=True)
            ismax = impT == mx
            first = jnp.min(jnp.where(ismax, blkT, nbk), axis=0, keepdims=True)
            onehot = blkT == first
            selT = selT | onehot
            impT = jnp.where(onehot, -2.0, impT)
        msel = _dot_tlhs(selT.astype(BF16), Ex)
        es = e * msel
        if ti == 0:
            es = jnp.where(causal, es, 0.0)
        es = es.astype(BF16)
        vv = jnp.concatenate([v, onescol], axis=1)
        oz = jnp.dot(es, vv, preferred_element_type=F32)
        o_sel = oz[:, :DH] / oz[:, DH:DH + 1]
        if ti == 0:
            ew = jnp.where(wmask, e, 0.0).astype(BF16)
            wz = jnp.dot(ew, vv, preferred_element_type=F32)
        else:
            kslab = k[row0 - 256:row0 + ATQ]
            vslab = v[row0 - 256:row0 + ATQ]
            sslab = _dot_trhs(q, kslab)
            ew = jnp.where(wmask, jnp.exp(sslab - c), 0.0).astype(BF16)
            vvs = jnp.concatenate([vslab, onescol[:WS]], axis=1)
            wz = jnp.dot(ew, vvs, preferred_element_type=F32)
        o_win = wz[:, :DH] / wz[:, DH:DH + 1]
        g = gsig[hh]
        outs.append((g[:, 0:1] * o_cmp + g[:, 1:2] * o_sel
                     + g[:, 2:3] * o_win).astype(BF16))
    o_ref[...] = jnp.concatenate(outs, axis=1)


def _attention_tile(q, k, v, gates_h, Wck, Wcv, ti):
    kw = (ti + 1) * ATQ
    nbk = kw // CBS
    body = functools.partial(_k3, ti=ti, kw=kw, nbk=nbk)
    return pl.pallas_call(
        body,
        grid=(HP,),
        in_specs=[
            pl.BlockSpec((ATQ, HG * DH), lambda h: (ti, h)),
            pl.BlockSpec((kw, HG * DH), lambda h: (0, h)),
            pl.BlockSpec((kw, HG * DH), lambda h: (0, h)),
            pl.BlockSpec((HG, ATQ, 3), lambda h: (h, ti, 0)),
            pl.BlockSpec((DH, DH), lambda h: (0, 0)),
            pl.BlockSpec((DH, DH), lambda h: (0, 0)),
        ],
        out_specs=pl.BlockSpec((ATQ, HG * DH), lambda h: (0, h)),
        out_shape=jax.ShapeDtypeStruct((ATQ, D), BF16),
    )(q, k, v, gates_h, Wck, Wcv)


def _k45(o_ref, x_ref, wo_ref, g_ref, b_ref, w1_ref, b1_ref, w2_ref, b2_ref,
         y_ref, wob_ref, w1b_ref, w2b_ref):
    @pl.when(pl.program_id(0) == 0)
    def _():
        wob_ref[...] = wo_ref[...].astype(BF16)
        w1b_ref[...] = w1_ref[...].astype(BF16)
        w2b_ref[...] = w2_ref[...].astype(BF16)
    x1 = x_ref[...] + jnp.dot(o_ref[...], wob_ref[...],
                              preferred_element_type=F32)
    xn = _ln_body(x1, g_ref[...], b_ref[...]).astype(BF16)
    hgelu = jax.nn.gelu(jnp.dot(xn, w1b_ref[...], preferred_element_type=F32)
                        + b1_ref[...])
    y_ref[...] = x1 + jnp.dot(hgelu.astype(BF16), w2b_ref[...],
                              preferred_element_type=F32) + b2_ref[...]


def _tail(o, x, Wo, ln_g, ln_b, W1, b1, W2, b2):
    return pl.pallas_call(
        _k45,
        grid=(NQT,),
        in_specs=[
            pl.BlockSpec((TQ, D), lambda i: (i, 0)),
            pl.BlockSpec((TQ, D), lambda i: (i, 0)),
            pl.BlockSpec((D, D), lambda i: (0, 0)),
            pl.BlockSpec((1, D), lambda i: (0, 0)),
            pl.BlockSpec((1, D), lambda i: (0, 0)),
            pl.BlockSpec((D, MLP), lambda i: (0, 0)),
            pl.BlockSpec((1, MLP), lambda i: (0, 0)),
            pl.BlockSpec((MLP, D), lambda i: (0, 0)),
            pl.BlockSpec((1, D), lambda i: (0, 0)),
        ],
        out_specs=pl.BlockSpec((TQ, D), lambda i: (i, 0)),
        out_shape=jax.ShapeDtypeStruct((S, D), F32),
        scratch_shapes=[
            pltpu.VMEM((D, D), BF16),
            pltpu.VMEM((D, MLP), BF16),
            pltpu.VMEM((MLP, D), BF16),
        ],
    )(o, x, Wo, ln_g, ln_b, W1, b1, W2, b2)


@jax.jit
def _run(x, ln1_g, ln1_b, Wq, Wk, Wv, Wck, Wcv, Wg, Wo, ln2_g, ln2_b, W1, b1, W2, b2):
    x2 = x[0]
    q, k, v, gates = _proj(x2, ln1_g[None], ln1_b[None], Wq, Wk, Wv,
                           Wg.astype(BF16))
    gates_h = gates.reshape(S, H, 3).transpose(1, 0, 2)
    Wckb = Wck.astype(BF16)
    Wcvb = Wcv.astype(BF16)
    o = jnp.concatenate(
        [_attention_tile(q, k, v, gates_h, Wckb, Wcvb, ti) for ti in range(4)],
        axis=0)
    y = _tail(o, x2, Wo, ln2_g[None], ln2_b[None], W1, b1[None], W2, b2[None])
    return y[None]


def kernel(x, ln1_g, ln1_b, Wq, Wk, Wv, Wck, Wcv, Wg, Wo, ln2_g, ln2_b, W1, b1, W2, b2):
    return _run(x, ln1_g, ln1_b, Wq, Wk, Wv, Wck, Wcv, Wg, Wo,
                ln2_g, ln2_b, W1, b1, W2, b2)
